# baseline (speedup 1.0000x reference)
import functools

import jax
import jax.numpy as jnp
from jax import lax
from jax.experimental import pallas as pl
from jax.experimental.pallas import tpu as pltpu
from jax.experimental.pallas import tpu_sc as plsc

NC = 2
NS = 16
LANES = 16


def _prep0_body(x_ref, w_ref, o_ref):
    o_ref[...] = jnp.dot(x_ref[...], w_ref[...],
                         preferred_element_type=jnp.float32)


def _tc_node_matmul(x, w, bn):
    n, d = x.shape
    return pl.pallas_call(
        _prep0_body,
        grid=(n // bn,),
        in_specs=[
            pl.BlockSpec((bn, d), lambda i: (i, 0)),
            pl.BlockSpec((d, w.shape[1]), lambda i: (0, 0)),
        ],
        out_specs=pl.BlockSpec((bn, w.shape[1]), lambda i: (i, 0)),
        out_shape=jax.ShapeDtypeStruct((n, w.shape[1]), jnp.float32),
    )(x, w)


def _edge_prep2_body(de, ea_ref, wee_ref, be_ref, eaw2_ref):
    ea = ea_ref[...][:, 0:de]
    eaw2_ref[...] = jnp.dot(ea, wee_ref[...],
                            preferred_element_type=jnp.float32) + be_ref[...]


def _edge_prep1_body(de, ea_ref, wme_ref, bm_ref, eaw_ref):
    ea = ea_ref[...][:, 0:de]
    eaw_ref[...] = jnp.dot(ea, wme_ref[...],
                           preferred_element_type=jnp.float32) + bm_ref[...]


def _edge_prep(ea, de, wme, bmv, be_blk):
    e, din = ea.shape
    d = wme.shape[1]
    return pl.pallas_call(
        functools.partial(_edge_prep1_body, de),
        grid=(e // be_blk,),
        in_specs=[
            pl.BlockSpec((be_blk, din), lambda i: (i, 0)),
            pl.BlockSpec((de, d), lambda i: (0, 0)),
            pl.BlockSpec((1, d), lambda i: (0, 0)),
        ],
        out_specs=pl.BlockSpec((be_blk, d), lambda i: (i, 0)),
        out_shape=jax.ShapeDtypeStruct((e, d), jnp.float32),
    )(ea, wme, bmv)


def _edge_prep1_alias_body(de, ea_ref, wme_ref, bm_ref, buf_ref, eaw_ref):
    del buf_ref
    ea = ea_ref[...][:, 0:de]
    eaw_ref[...] = jnp.dot(ea, wme_ref[...],
                           preferred_element_type=jnp.float32) + bm_ref[...]


def _edge_prep_half(ea_half, de, wme, bmv, be_blk, e_total, half_idx,
                    eaw_buf=None):
    eh, din = ea_half.shape
    d = wme.shape[1]
    nb = eh // be_blk
    off = half_idx * nb
    if eaw_buf is None:
        return pl.pallas_call(
            functools.partial(_edge_prep1_body, de),
            grid=(nb,),
            in_specs=[
                pl.BlockSpec((be_blk, din), lambda i: (i, 0)),
                pl.BlockSpec((de, d), lambda i: (0, 0)),
                pl.BlockSpec((1, d), lambda i: (0, 0)),
            ],
            out_specs=pl.BlockSpec((be_blk, d), lambda i: (i + off, 0)),
            out_shape=jax.ShapeDtypeStruct((e_total, d), jnp.float32),
        )(ea_half, wme, bmv)
    return pl.pallas_call(
        functools.partial(_edge_prep1_alias_body, de),
        grid=(nb,),
        in_specs=[
            pl.BlockSpec((be_blk, din), lambda i: (i, 0)),
            pl.BlockSpec((de, d), lambda i: (0, 0)),
            pl.BlockSpec((1, d), lambda i: (0, 0)),
            pl.BlockSpec((be_blk, d), lambda i: (i + off, 0)),
        ],
        out_specs=pl.BlockSpec((be_blk, d), lambda i: (i + off, 0)),
        out_shape=jax.ShapeDtypeStruct((e_total, d), jnp.float32),
        input_output_aliases={3: 0},
    )(ea_half, wme, bmv, eaw_buf)


def _edge_prep2_alias_body(de, ea_ref, wee_ref, be_ref, buf_ref, eaw2_ref):
    del buf_ref
    ea = ea_ref[...][:, 0:de]
    eaw2_ref[...] = jnp.dot(ea, wee_ref[...],
                            preferred_element_type=jnp.float32) + be_ref[...]


def _edge_prep2_half(ea_half, de, wee_p, bev_p, be_blk, e_total, half_idx,
                     buf=None):
    eh, din = ea_half.shape
    dp = wee_p.shape[1]
    nb = eh // be_blk
    off = half_idx * nb
    if buf is None:
        return pl.pallas_call(
            functools.partial(_edge_prep2_body, de),
            grid=(nb,),
            in_specs=[
                pl.BlockSpec((be_blk, din), lambda i: (i, 0)),
                pl.BlockSpec((de, dp), lambda i: (0, 0)),
                pl.BlockSpec((1, dp), lambda i: (0, 0)),
            ],
            out_specs=pl.BlockSpec((be_blk, dp), lambda i: (i + off, 0)),
            out_shape=jax.ShapeDtypeStruct((e_total, dp), jnp.float32),
        )(ea_half, wee_p, bev_p)
    return pl.pallas_call(
        functools.partial(_edge_prep2_alias_body, de),
        grid=(nb,),
        in_specs=[
            pl.BlockSpec((be_blk, din), lambda i: (i, 0)),
            pl.BlockSpec((de, dp), lambda i: (0, 0)),
            pl.BlockSpec((1, dp), lambda i: (0, 0)),
            pl.BlockSpec((be_blk, dp), lambda i: (i + off, 0)),
        ],
        out_specs=pl.BlockSpec((be_blk, dp), lambda i: (i + off, 0)),
        out_shape=jax.ShapeDtypeStruct((e_total, dp), jnp.float32),
        input_output_aliases={3: 0},
    )(ea_half, wee_p, bev_p, buf)


def _edge_prep2(ea, de, wee_p, bev_p, be_blk):
    e, din = ea.shape
    dp = wee_p.shape[1]
    return pl.pallas_call(
        functools.partial(_edge_prep2_body, de),
        grid=(e // be_blk,),
        in_specs=[
            pl.BlockSpec((be_blk, din), lambda i: (i, 0)),
            pl.BlockSpec((de, dp), lambda i: (0, 0)),
            pl.BlockSpec((1, dp), lambda i: (0, 0)),
        ],
        out_specs=pl.BlockSpec((be_blk, dp), lambda i: (i, 0)),
        out_shape=jax.ShapeDtypeStruct((e, dp), jnp.float32),
    )(ea, wee_p, bev_p)


def _update2_body(sp_ref, cp_ref, h_ref, waa_ref, wah_ref, ba_ref,
                  wmxn_ref, wij_ref,
                  hn_ref, hxn_ref, hij_ref):
    s = sp_ref[0] + sp_ref[1]
    cnt = cp_ref[0, :, 0:1] + cp_ref[1, :, 0:1]
    agg = s * (1.0 / jnp.maximum(cnt, 1.0))
    u = jnp.dot(agg, waa_ref[...], preferred_element_type=jnp.float32)
    u = u + jnp.dot(h_ref[...], wah_ref[...],
                    preferred_element_type=jnp.float32)
    u = jnp.maximum(u + ba_ref[...], 0.0)
    nn = jnp.sqrt(jnp.sum(u * u, axis=1, keepdims=True))
    hv = u / jnp.maximum(nn, 1e-12)
    hn_ref[...] = hv
    hxn_ref[...] = jnp.dot(hv, wmxn_ref[...],
                           preferred_element_type=jnp.float32)
    hij_ref[...] = jnp.dot(hv, wij_ref[...],
                           preferred_element_type=jnp.float32)


def _update1_body(sp_ref, cp_ref, h_ref, waa_ref, wah_ref, ba_ref, hn_ref):
    s = sp_ref[0] + sp_ref[1]
    cnt = cp_ref[0, :, 0:1] + cp_ref[1, :, 0:1]
    agg = s * (1.0 / jnp.maximum(cnt, 1.0))
    u = jnp.dot(agg, waa_ref[...], preferred_element_type=jnp.float32)
    u = u + jnp.dot(h_ref[...], wah_ref[...],
                    preferred_element_type=jnp.float32)
    u = jnp.maximum(u + ba_ref[...], 0.0)
    nn = jnp.sqrt(jnp.sum(u * u, axis=1, keepdims=True))
    hn_ref[...] = u / jnp.maximum(nn, 1e-12)


def _update(sp, cp, h, waa, wah, bav, wmxn, wij_p, bn):
    n, d = h.shape
    de = cp.shape[2]
    grid = (n // bn,)
    common_in = [
        pl.BlockSpec((NC, bn, d), lambda i: (0, i, 0)),
        pl.BlockSpec((NC, bn, de), lambda i: (0, i, 0)),
        pl.BlockSpec((bn, d), lambda i: (i, 0)),
        pl.BlockSpec((d, d), lambda i: (0, 0)),
        pl.BlockSpec((d, d), lambda i: (0, 0)),
        pl.BlockSpec((1, d), lambda i: (0, 0)),
    ]
    if wmxn is None:
        return pl.pallas_call(
            _update1_body,
            grid=grid,
            in_specs=common_in,
            out_specs=pl.BlockSpec((bn, d), lambda i: (i, 0)),
            out_shape=jax.ShapeDtypeStruct((n, d), jnp.float32),
        )(sp, cp, h, waa, wah, bav)
    dp = wij_p.shape[1]
    return pl.pallas_call(
        _update2_body,
        grid=grid,
        in_specs=common_in + [
            pl.BlockSpec((d, d), lambda i: (0, 0)),
            pl.BlockSpec((d, dp), lambda i: (0, 0)),
        ],
        out_specs=[
            pl.BlockSpec((bn, d), lambda i: (i, 0)),
            pl.BlockSpec((bn, d), lambda i: (i, 0)),
            pl.BlockSpec((bn, dp), lambda i: (i, 0)),
        ],
        out_shape=[
            jax.ShapeDtypeStruct((n, d), jnp.float32),
            jax.ShapeDtypeStruct((n, d), jnp.float32),
            jax.ShapeDtypeStruct((n, dp), jnp.float32),
        ],
    )(sp, cp, h, waa, wah, bav, wmxn, wij_p)


def _padded_rows(nn):
    rpt = -(-nn // NS)
    rpt = -(-rpt // 128) * 128
    return rpt, rpt * NS


def _make_msg_kernel(nn, dd, ee):
    w = NC * NS
    ept = ee // w
    ch = 40
    nchunk = ept // ch
    rpt, nnp = _padded_rows(nn)
    nz = rpt // ch
    assert ept % ch == 0 and rpt % ch == 0 and dd % LANES == 0
    assert nchunk % 2 == 0 and nchunk >= 8

    mesh = plsc.VectorSubcoreMesh(core_axis_name="c", subcore_axis_name="s",
                                  num_cores=NC, num_subcores=NS)

    out_type = jax.ShapeDtypeStruct((NC, nnp, dd), jnp.float32)
    scratch = (
        [pltpu.VMEM((8, 2, ch), jnp.int32),
         pltpu.VMEM((4, ch, dd), jnp.float32),
         pltpu.VMEM((2, ch, dd), jnp.float32),
         pltpu.VMEM_SHARED((nnp, dd), jnp.float32)]
        + [pltpu.SemaphoreType.DMA] * 10
    )

    def body(hx, eaw, srcr, dstr, out_s, ibuf, rows, eawb, acc, *sem):
        c = lax.axis_index("c")
        s = lax.axis_index("s")
        ncol = dd // LANES
        semi = list(sem[0:4])
        semg = list(sem[4:6])
        seme = list(sem[6:8])
        sems = list(sem[8:10])

        def zrow(r, carry):
            for cc in range(ncol):
                rows[0, r, cc * LANES:(cc + 1) * LANES] = jnp.zeros(
                    (LANES,), jnp.float32)
            return carry
        lax.fori_loop(0, ch, zrow, 0)

        base_row = s * rpt
        for z in range(nz):
            pltpu.sync_copy(rows.at[0], acc.at[pl.ds(base_row + z * ch, ch)])

        plsc.subcore_barrier()

        ebase = (c * NS + s) * ept

        def issue_idx(j, s8):
            boff = ebase + j * ch
            pltpu.async_copy(srcr.at[pl.ds(boff, ch)], ibuf.at[s8, 0],
                             semi[s8 % 4])
            pltpu.async_copy(dstr.at[pl.ds(boff, ch)], ibuf.at[s8, 1],
                             semi[s8 % 4])

        def wait_idx(j, s8):
            boff = ebase + j * ch
            pltpu.make_async_copy(srcr.at[pl.ds(boff, ch)], ibuf.at[s8, 0],
                                  semi[s8 % 4]).wait()
            pltpu.make_async_copy(dstr.at[pl.ds(boff, ch)], ibuf.at[s8, 1],
                                  semi[s8 % 4]).wait()

        def process(j, s8, pf_idx, pf_g, drain):
            b = s8 % 2
            s4 = s8 % 4
            boff = ebase + j * ch
            pltpu.make_async_copy(hx.at[ibuf.at[s8, 0]], rows.at[s4],
                                  semg[b]).wait()
            pltpu.make_async_copy(eaw.at[pl.ds(boff, ch)], eawb.at[b],
                                  seme[b]).wait()

            def crow(r, carry2):
                for cc in range(ncol):
                    sl = slice(cc * LANES, (cc + 1) * LANES)
                    rows[s4, r, sl] = jnp.maximum(
                        rows[s4, r, sl] + eawb[b, r, sl], 0.0)
                return carry2
            lax.fori_loop(0, ch, crow, 0)

            if drain:
                pltpu.make_async_copy(
                    rows.at[(s4 + 2) % 4], acc.at[ibuf.at[(s8 + 6) % 8, 1]],
                    sems[b]).wait()
            pltpu.async_copy(rows.at[s4], acc.at[ibuf.at[s8, 1]],
                             sems[b], add=True)
            if pf_g:
                pltpu.async_copy(eaw.at[pl.ds(boff + 2 * ch, ch)],
                                 eawb.at[b], seme[b])
            if pf_idx:
                issue_idx(j + 4, (s8 + 4) % 8)
            if pf_g:
                wait_idx(j + 2, (s8 + 2) % 8)
                pltpu.async_copy(hx.at[ibuf.at[(s8 + 2) % 8, 0]],
                                 rows.at[(s4 + 2) % 4], semg[b])

        for j in range(4):
            issue_idx(j, j)
        for j in range(2):
            wait_idx(j, j)
            pltpu.async_copy(hx.at[ibuf.at[j, 0]], rows.at[j], semg[j])
            pltpu.async_copy(eaw.at[pl.ds(ebase + j * ch, ch)],
                             eawb.at[j], seme[j])

        tail_start = ((nchunk - 4) // 8) * 8
        for j in range(8):
            process(j, j, True, True, j >= 2)

        def step(g, carry):
            for b8 in range(8):
                process(8 * g + b8, b8, True, True, True)
            return carry
        lax.fori_loop(1, tail_start // 8, step, 0)
        for j in range(tail_start, nchunk):
            process(j, j % 8, j + 4 < nchunk, j + 2 < nchunk, True)
        for j in (nchunk - 2, nchunk - 1):
            pltpu.make_async_copy(
                rows.at[j % 4], acc.at[ibuf.at[j % 8, 1]],
                sems[j % 2]).wait()

        plsc.subcore_barrier()
        pltpu.sync_copy(acc.at[pl.ds(base_row, rpt)],
                        out_s.at[c, pl.ds(base_row, rpt)])

    return pl.kernel(body, out_type=out_type, mesh=mesh,
                     scratch_types=scratch)


def _make_cnt_kernel(nn, ee, dd):
    w = NC * NS
    ept = ee // w
    ch = 40
    nchunk = ept // ch
    rpt, nnp = _padded_rows(nn)
    nz = rpt // ch
    assert ept % ch == 0 and rpt % ch == 0
    assert nchunk % 2 == 0 and nchunk >= 8

    mesh = plsc.VectorSubcoreMesh(core_axis_name="c", subcore_axis_name="s",
                                  num_cores=NC, num_subcores=NS)
    out_type = jax.ShapeDtypeStruct((NC, nnp, dd), jnp.float32)
    scratch = (
        [pltpu.VMEM((8, 1, ch), jnp.int32),
         pltpu.VMEM((ch, dd), jnp.float32),
         pltpu.VMEM((ch, dd), jnp.float32),
         pltpu.VMEM_SHARED((nnp, dd), jnp.float32)]
        + [pltpu.SemaphoreType.DMA] * 6
    )

    def body(dstr, out_c, dsti, ones, zbuf, acccnt, *sem):
        c = lax.axis_index("c")
        s = lax.axis_index("s")
        semi = list(sem[0:4])
        sems = list(sem[4:6])

        def fill(r, carry):
            for cc in range(dd // LANES):
                sl = slice(cc * LANES, (cc + 1) * LANES)
                ones[r, sl] = jnp.ones((LANES,), jnp.float32)
                zbuf[r, sl] = jnp.zeros((LANES,), jnp.float32)
            return carry
        lax.fori_loop(0, ch, fill, 0)

        base_row = s * rpt
        for z in range(nz):
            pltpu.sync_copy(zbuf, acccnt.at[pl.ds(base_row + z * ch, ch)])

        plsc.subcore_barrier()

        ebase = (c * NS + s) * ept

        def issue_idx(j, s8):
            pltpu.async_copy(dstr.at[pl.ds(ebase + j * ch, ch)],
                             dsti.at[s8, 0], semi[s8 % 4])

        def process(j, s8, pf_idx, drain):
            b = s8 % 2
            pltpu.make_async_copy(dstr.at[pl.ds(ebase + j * ch, ch)],
                                  dsti.at[s8, 0], semi[s8 % 4]).wait()
            if drain:
                pltpu.make_async_copy(ones, acccnt.at[dsti.at[(s8 + 6) % 8, 0]],
                                      sems[b]).wait()
            pltpu.async_copy(ones, acccnt.at[dsti.at[s8, 0]], sems[b],
                             add=True)
            if pf_idx:
                issue_idx(j + 4, (s8 + 4) % 8)

        for j in range(4):
            issue_idx(j, j)

        tail_start = ((nchunk - 4) // 8) * 8
        for j in range(8):
            process(j, j, True, j >= 2)

        def step(g, carry):
            for b8 in range(8):
                process(8 * g + b8, b8, True, True)
            return carry
        lax.fori_loop(1, tail_start // 8, step, 0)
        for j in range(tail_start, nchunk):
            process(j, j % 8, j + 4 < nchunk, True)
        for j in (nchunk - 2, nchunk - 1):
            pltpu.make_async_copy(ones, acccnt.at[dsti.at[j % 8, 0]],
                                  sems[j % 2]).wait()

        plsc.subcore_barrier()
        pltpu.sync_copy(acccnt.at[pl.ds(base_row, rpt)],
                        out_c.at[c, pl.ds(base_row, rpt)])

    return pl.kernel(body, out_type=out_type, mesh=mesh,
                     scratch_types=scratch)


def _make_edge_update_kernel(nn, dp, ee, e0, esz):
    w = NC * NS
    ept = esz // w
    ch = 40
    nchunk = ept // ch
    assert ept % ch == 0
    assert nchunk >= 8

    mesh = plsc.VectorSubcoreMesh(core_axis_name="c", subcore_axis_name="s",
                                  num_cores=NC, num_subcores=NS)
    out_type = jax.ShapeDtypeStruct((esz, dp), jnp.float32)
    scratch = (
        [pltpu.VMEM((8, 2, ch), jnp.int32),
         pltpu.VMEM((4, ch, dp), jnp.float32),
         pltpu.VMEM((4, ch, dp), jnp.float32),
         pltpu.VMEM((4, ch, dp), jnp.float32)]
        + [pltpu.SemaphoreType.DMA] * 12
    )

    def body(hij, eaw2, srcr, dstr, out, ibuf, g1, g2, eb, *sem):
        c = lax.axis_index("c")
        s = lax.axis_index("s")
        semi = list(sem[0:4])
        sem1 = list(sem[4:6])
        sem2 = list(sem[6:8])
        seme = list(sem[8:10])
        semo = list(sem[10:12])
        ebase = (c * NS + s) * ept
        gbase = e0 + ebase

        def issue_idx(j, s8):
            boff = gbase + j * ch
            pltpu.async_copy(srcr.at[pl.ds(boff, ch)], ibuf.at[s8, 0],
                             semi[s8 % 4])
            pltpu.async_copy(dstr.at[pl.ds(boff, ch)], ibuf.at[s8, 1],
                             semi[s8 % 4])

        def wait_idx(j, s8):
            boff = gbase + j * ch
            pltpu.make_async_copy(srcr.at[pl.ds(boff, ch)],
                                  ibuf.at[s8, 0], semi[s8 % 4]).wait()
            pltpu.make_async_copy(dstr.at[pl.ds(boff, ch)],
                                  ibuf.at[s8, 1], semi[s8 % 4]).wait()

        def issue_loads(j, s8):
            boff = gbase + j * ch
            s4 = s8 % 4
            pltpu.async_copy(hij.at[ibuf.at[s8, 0]], g1.at[s4],
                             sem1[s8 % 2])
            pltpu.async_copy(hij.at[ibuf.at[s8, 1]], g2.at[s4],
                             sem2[s8 % 2])
            pltpu.async_copy(eaw2.at[pl.ds(boff, ch)], eb.at[s4],
                             seme[s8 % 2])

        def process(j, s8, pf_idx, pf_g, drain):
            b = s8 % 2
            s4 = s8 % 4
            boff = ebase + j * ch
            goff = gbase + j * ch
            pltpu.make_async_copy(hij.at[ibuf.at[s8, 0]], g1.at[s4],
                                  sem1[b]).wait()
            pltpu.make_async_copy(hij.at[ibuf.at[s8, 1]], g2.at[s4],
                                  sem2[b]).wait()
            pltpu.make_async_copy(eaw2.at[pl.ds(goff, ch)], eb.at[s4],
                                  seme[b]).wait()

            def crow(r, carry2):
                v = (eb[s4, r, 0:LANES] + g1[s4, r, 0:LANES]
                     + g2[s4, r, LANES:2 * LANES])
                eb[s4, r, 0:LANES] = jnp.maximum(v, 0.0)
                return carry2
            lax.fori_loop(0, ch, crow, 0)

            if drain:
                pltpu.make_async_copy(
                    eb.at[(s4 + 2) % 4],
                    out.at[pl.ds(boff - 2 * ch, ch)], semo[b]).wait()
            pltpu.async_copy(eb.at[s4], out.at[pl.ds(boff, ch)], semo[b])
            if pf_idx:
                issue_idx(j + 4, (s8 + 4) % 8)
            if pf_g:
                wait_idx(j + 2, (s8 + 2) % 8)
                issue_loads(j + 2, (s8 + 2) % 8)

        for j in range(4):
            issue_idx(j, j)
        for j in range(2):
            wait_idx(j, j)
            issue_loads(j, j)

        tail_start = ((nchunk - 4) // 8) * 8
        for j in range(8):
            process(j, j, True, True, j >= 2)

        def step(g, carry):
            for b8 in range(8):
                process(8 * g + b8, b8, True, True, True)
            return carry
        lax.fori_loop(1, tail_start // 8, step, 0)
        for j in range(tail_start, nchunk):
            process(j, j % 8, j + 4 < nchunk, j + 2 < nchunk, True)
        for j in (nchunk - 2, nchunk - 1):
            pltpu.make_async_copy(
                eb.at[j % 4], out.at[pl.ds(ebase + j * ch, ch)],
                semo[j % 2]).wait()

    return pl.kernel(body, out_type=out_type, mesh=mesh,
                     scratch_types=scratch)


def kernel(x, edge_attr, edge_index, Wm, bm, Wa, ba, We, be):
    n, d = x.shape
    e, de = edge_attr.shape
    nl = Wm.shape[0]
    assert de == LANES

    src = edge_index[0].astype(jnp.int32)
    dst = edge_index[1].astype(jnp.int32)

    bn = 400
    be_blk = 2000

    dp = 128

    msg = _make_msg_kernel(n, d, e)
    cntk = _make_cnt_kernel(n, e, d)
    eh = e // 2
    edge_upd_a = _make_edge_update_kernel(n, dp, e, 0, eh)
    edge_upd_b = _make_edge_update_kernel(n, dp, e, eh, eh)

    h = x
    ea_halves = (edge_attr[:eh], edge_attr[eh:])
    hx = _tc_node_matmul(x, Wm[0][:d], bn)
    cp = cntk(dst)
    for l in range(nl):
        last = l == nl - 1
        eaw = _edge_prep_half(ea_halves[0], de, Wm[l][d:], bm[l][None],
                              be_blk, e, 0)
        eaw = _edge_prep_half(ea_halves[1], de, Wm[l][d:], bm[l][None],
                              be_blk, e, 1, eaw)
        if not last:
            wee_p = jnp.pad(We[l][2 * d:], ((0, 0), (0, dp - de)))
            bev_p = jnp.pad(be[l], (0, dp - de))[None]
            eaw2 = _edge_prep2_half(ea_halves[0], de, wee_p, bev_p,
                                    be_blk, e, 0)
            eaw2 = _edge_prep2_half(ea_halves[1], de, wee_p, bev_p,
                                    be_blk, e, 1, eaw2)
        sp = msg(hx, eaw, src, dst)
        if not last:
            wij_p = jnp.pad(
                jnp.concatenate([We[l][:d], We[l][d:2 * d]], axis=1),
                ((0, 0), (0, dp - 2 * de)))
            h, hx, hij = _update(sp, cp, h, Wa[l][:d], Wa[l][d:],
                                 ba[l][None], Wm[l + 1][:d], wij_p, bn)
            ea_halves = (edge_upd_a(hij, eaw2, src, dst),
                         edge_upd_b(hij, eaw2, src, dst))
        else:
            h = _update(sp, cp, h, Wa[l][:d], Wa[l][d:], ba[l][None],
                        None, None, bn)
    return h


# speedup vs baseline: 1.0170x; 1.0170x over previous
import functools

import jax
import jax.numpy as jnp
from jax import lax
from jax.experimental import pallas as pl
from jax.experimental.pallas import tpu as pltpu
from jax.experimental.pallas import tpu_sc as plsc

NC = 2
NS = 16
LANES = 16


def _prep0_body(x_ref, w_ref, o_ref):
    o_ref[...] = jnp.dot(x_ref[...], w_ref[...],
                         preferred_element_type=jnp.float32)


def _tc_node_matmul(x, w, bn):
    n, d = x.shape
    return pl.pallas_call(
        _prep0_body,
        grid=(n // bn,),
        in_specs=[
            pl.BlockSpec((bn, d), lambda i: (i, 0)),
            pl.BlockSpec((d, w.shape[1]), lambda i: (0, 0)),
        ],
        out_specs=pl.BlockSpec((bn, w.shape[1]), lambda i: (i, 0)),
        out_shape=jax.ShapeDtypeStruct((n, w.shape[1]), jnp.float32),
    )(x, w)


def _edge_prep2_body(de, ea_ref, wee_ref, be_ref, eaw2_ref):
    ea = ea_ref[...][:, 0:de]
    eaw2_ref[...] = jnp.dot(ea, wee_ref[...],
                            preferred_element_type=jnp.float32) + be_ref[...]


def _edge_prep1_body(de, ea_ref, wme_ref, bm_ref, eaw_ref):
    ea = ea_ref[...][:, 0:de]
    eaw_ref[...] = jnp.dot(ea, wme_ref[...],
                           preferred_element_type=jnp.float32) + bm_ref[...]


def _edge_prep(ea, de, wme, bmv, be_blk):
    e, din = ea.shape
    d = wme.shape[1]
    return pl.pallas_call(
        functools.partial(_edge_prep1_body, de),
        grid=(e // be_blk,),
        in_specs=[
            pl.BlockSpec((be_blk, din), lambda i: (i, 0)),
            pl.BlockSpec((de, d), lambda i: (0, 0)),
            pl.BlockSpec((1, d), lambda i: (0, 0)),
        ],
        out_specs=pl.BlockSpec((be_blk, d), lambda i: (i, 0)),
        out_shape=jax.ShapeDtypeStruct((e, d), jnp.float32),
    )(ea, wme, bmv)


def _edge_prep1_alias_body(de, ea_ref, wme_ref, bm_ref, buf_ref, eaw_ref):
    del buf_ref
    ea = ea_ref[...][:, 0:de]
    eaw_ref[...] = jnp.dot(ea, wme_ref[...],
                           preferred_element_type=jnp.float32) + bm_ref[...]


def _edge_prep_half(ea_half, de, wme, bmv, be_blk, e_total, half_idx,
                    eaw_buf=None):
    eh, din = ea_half.shape
    d = wme.shape[1]
    nb = eh // be_blk
    off = half_idx * nb
    if eaw_buf is None:
        return pl.pallas_call(
            functools.partial(_edge_prep1_body, de),
            grid=(nb,),
            in_specs=[
                pl.BlockSpec((be_blk, din), lambda i: (i, 0)),
                pl.BlockSpec((de, d), lambda i: (0, 0)),
                pl.BlockSpec((1, d), lambda i: (0, 0)),
            ],
            out_specs=pl.BlockSpec((be_blk, d), lambda i: (i + off, 0)),
            out_shape=jax.ShapeDtypeStruct((e_total, d), jnp.float32),
        )(ea_half, wme, bmv)
    return pl.pallas_call(
        functools.partial(_edge_prep1_alias_body, de),
        grid=(nb,),
        in_specs=[
            pl.BlockSpec((be_blk, din), lambda i: (i, 0)),
            pl.BlockSpec((de, d), lambda i: (0, 0)),
            pl.BlockSpec((1, d), lambda i: (0, 0)),
            pl.BlockSpec((be_blk, d), lambda i: (i + off, 0)),
        ],
        out_specs=pl.BlockSpec((be_blk, d), lambda i: (i + off, 0)),
        out_shape=jax.ShapeDtypeStruct((e_total, d), jnp.float32),
        input_output_aliases={3: 0},
    )(ea_half, wme, bmv, eaw_buf)


def _edge_prep2_alias_body(de, ea_ref, wee_ref, be_ref, buf_ref, eaw2_ref):
    del buf_ref
    ea = ea_ref[...][:, 0:de]
    eaw2_ref[...] = jnp.dot(ea, wee_ref[...],
                            preferred_element_type=jnp.float32) + be_ref[...]


def _edge_prep2_half(ea_half, de, wee_p, bev_p, be_blk, e_total, half_idx,
                     buf=None):
    eh, din = ea_half.shape
    dp = wee_p.shape[1]
    nb = eh // be_blk
    off = half_idx * nb
    if buf is None:
        return pl.pallas_call(
            functools.partial(_edge_prep2_body, de),
            grid=(nb,),
            in_specs=[
                pl.BlockSpec((be_blk, din), lambda i: (i, 0)),
                pl.BlockSpec((de, dp), lambda i: (0, 0)),
                pl.BlockSpec((1, dp), lambda i: (0, 0)),
            ],
            out_specs=pl.BlockSpec((be_blk, dp), lambda i: (i + off, 0)),
            out_shape=jax.ShapeDtypeStruct((e_total, dp), jnp.float32),
        )(ea_half, wee_p, bev_p)
    return pl.pallas_call(
        functools.partial(_edge_prep2_alias_body, de),
        grid=(nb,),
        in_specs=[
            pl.BlockSpec((be_blk, din), lambda i: (i, 0)),
            pl.BlockSpec((de, dp), lambda i: (0, 0)),
            pl.BlockSpec((1, dp), lambda i: (0, 0)),
            pl.BlockSpec((be_blk, dp), lambda i: (i + off, 0)),
        ],
        out_specs=pl.BlockSpec((be_blk, dp), lambda i: (i + off, 0)),
        out_shape=jax.ShapeDtypeStruct((e_total, dp), jnp.float32),
        input_output_aliases={3: 0},
    )(ea_half, wee_p, bev_p, buf)


def _edge_prep2(ea, de, wee_p, bev_p, be_blk):
    e, din = ea.shape
    dp = wee_p.shape[1]
    return pl.pallas_call(
        functools.partial(_edge_prep2_body, de),
        grid=(e // be_blk,),
        in_specs=[
            pl.BlockSpec((be_blk, din), lambda i: (i, 0)),
            pl.BlockSpec((de, dp), lambda i: (0, 0)),
            pl.BlockSpec((1, dp), lambda i: (0, 0)),
        ],
        out_specs=pl.BlockSpec((be_blk, dp), lambda i: (i, 0)),
        out_shape=jax.ShapeDtypeStruct((e, dp), jnp.float32),
    )(ea, wee_p, bev_p)


def _update2_body(sp_ref, cp_ref, h_ref, waa_ref, wah_ref, ba_ref,
                  wmxn_ref, wij_ref,
                  hn_ref, hxn_ref, hij_ref):
    s = sp_ref[0] + sp_ref[1]
    cnt = cp_ref[0, :, 0:1] + cp_ref[1, :, 0:1]
    agg = s * (1.0 / jnp.maximum(cnt, 1.0))
    u = jnp.dot(agg, waa_ref[...], preferred_element_type=jnp.float32)
    u = u + jnp.dot(h_ref[...], wah_ref[...],
                    preferred_element_type=jnp.float32)
    u = jnp.maximum(u + ba_ref[...], 0.0)
    nn = jnp.sqrt(jnp.sum(u * u, axis=1, keepdims=True))
    hv = u / jnp.maximum(nn, 1e-12)
    hn_ref[...] = hv
    hxn_ref[...] = jnp.dot(hv, wmxn_ref[...],
                           preferred_element_type=jnp.float32)
    hij_ref[...] = jnp.dot(hv, wij_ref[...],
                           preferred_element_type=jnp.float32)


def _update1_body(sp_ref, cp_ref, h_ref, waa_ref, wah_ref, ba_ref, hn_ref):
    s = sp_ref[0] + sp_ref[1]
    cnt = cp_ref[0, :, 0:1] + cp_ref[1, :, 0:1]
    agg = s * (1.0 / jnp.maximum(cnt, 1.0))
    u = jnp.dot(agg, waa_ref[...], preferred_element_type=jnp.float32)
    u = u + jnp.dot(h_ref[...], wah_ref[...],
                    preferred_element_type=jnp.float32)
    u = jnp.maximum(u + ba_ref[...], 0.0)
    nn = jnp.sqrt(jnp.sum(u * u, axis=1, keepdims=True))
    hn_ref[...] = u / jnp.maximum(nn, 1e-12)


def _update(sp, cp, h, waa, wah, bav, wmxn, wij_p, bn):
    n, d = h.shape
    de = cp.shape[2]
    grid = (n // bn,)
    common_in = [
        pl.BlockSpec((NC, bn, d), lambda i: (0, i, 0)),
        pl.BlockSpec((NC, bn, de), lambda i: (0, i, 0)),
        pl.BlockSpec((bn, d), lambda i: (i, 0)),
        pl.BlockSpec((d, d), lambda i: (0, 0)),
        pl.BlockSpec((d, d), lambda i: (0, 0)),
        pl.BlockSpec((1, d), lambda i: (0, 0)),
    ]
    if wmxn is None:
        return pl.pallas_call(
            _update1_body,
            grid=grid,
            in_specs=common_in,
            out_specs=pl.BlockSpec((bn, d), lambda i: (i, 0)),
            out_shape=jax.ShapeDtypeStruct((n, d), jnp.float32),
        )(sp, cp, h, waa, wah, bav)
    dp = wij_p.shape[1]
    return pl.pallas_call(
        _update2_body,
        grid=grid,
        in_specs=common_in + [
            pl.BlockSpec((d, d), lambda i: (0, 0)),
            pl.BlockSpec((d, dp), lambda i: (0, 0)),
        ],
        out_specs=[
            pl.BlockSpec((bn, d), lambda i: (i, 0)),
            pl.BlockSpec((bn, d), lambda i: (i, 0)),
            pl.BlockSpec((bn, dp), lambda i: (i, 0)),
        ],
        out_shape=[
            jax.ShapeDtypeStruct((n, d), jnp.float32),
            jax.ShapeDtypeStruct((n, d), jnp.float32),
            jax.ShapeDtypeStruct((n, dp), jnp.float32),
        ],
    )(sp, cp, h, waa, wah, bav, wmxn, wij_p)


def _padded_rows(nn):
    rpt = -(-nn // NS)
    rpt = -(-rpt // 128) * 128
    return rpt, rpt * NS


def _make_msg_kernel(nn, dd, ee):
    w = NC * NS
    ept = ee // w
    ch = 40
    nchunk = ept // ch
    rpt, nnp = _padded_rows(nn)
    nz = rpt // ch
    assert ept % ch == 0 and rpt % ch == 0 and dd % LANES == 0
    assert nchunk % 2 == 0 and nchunk >= 8

    mesh = plsc.VectorSubcoreMesh(core_axis_name="c", subcore_axis_name="s",
                                  num_cores=NC, num_subcores=NS)

    out_type = jax.ShapeDtypeStruct((NC, nnp, dd), jnp.float32)
    scratch = (
        [pltpu.VMEM((8, 2, ch), jnp.int32),
         pltpu.VMEM((4, ch, dd), jnp.float32),
         pltpu.VMEM((2, ch, dd), jnp.float32),
         pltpu.VMEM_SHARED((nnp, dd), jnp.float32)]
        + [pltpu.SemaphoreType.DMA] * 10
    )

    def body(hx, eaw, srcr, dstr, out_s, ibuf, rows, eawb, acc, *sem):
        c = lax.axis_index("c")
        s = lax.axis_index("s")
        ncol = dd // LANES
        semi = list(sem[0:4])
        semg = list(sem[4:6])
        seme = list(sem[6:8])
        sems = list(sem[8:10])

        def zrow(r, carry):
            for cc in range(ncol):
                rows[0, r, cc * LANES:(cc + 1) * LANES] = jnp.zeros(
                    (LANES,), jnp.float32)
            return carry
        lax.fori_loop(0, ch, zrow, 0)

        base_row = s * rpt
        for z in range(nz):
            pltpu.sync_copy(rows.at[0], acc.at[pl.ds(base_row + z * ch, ch)])

        plsc.subcore_barrier()

        ebase = (c * NS + s) * ept

        def issue_idx(j, s8):
            boff = ebase + j * ch
            pltpu.async_copy(srcr.at[pl.ds(boff, ch)], ibuf.at[s8, 0],
                             semi[s8 % 4])
            pltpu.async_copy(dstr.at[pl.ds(boff, ch)], ibuf.at[s8, 1],
                             semi[s8 % 4])

        def wait_idx(j, s8):
            boff = ebase + j * ch
            pltpu.make_async_copy(srcr.at[pl.ds(boff, ch)], ibuf.at[s8, 0],
                                  semi[s8 % 4]).wait()
            pltpu.make_async_copy(dstr.at[pl.ds(boff, ch)], ibuf.at[s8, 1],
                                  semi[s8 % 4]).wait()

        def process(j, s8, pf_idx, pf_g, drain):
            b = s8 % 2
            s4 = s8 % 4
            boff = ebase + j * ch
            pltpu.make_async_copy(hx.at[ibuf.at[s8, 0]], rows.at[s4],
                                  semg[b]).wait()
            pltpu.make_async_copy(eaw.at[pl.ds(boff, ch)], eawb.at[b],
                                  seme[b]).wait()
            if drain:
                pltpu.make_async_copy(
                    rows.at[(s4 + 2) % 4], acc.at[ibuf.at[(s8 + 6) % 8, 1]],
                    sems[b]).wait()
            if pf_g:
                wait_idx(j + 2, (s8 + 2) % 8)
                pltpu.async_copy(hx.at[ibuf.at[(s8 + 2) % 8, 0]],
                                 rows.at[(s4 + 2) % 4], semg[b])

            def crow(r, carry2):
                for cc in range(ncol):
                    sl = slice(cc * LANES, (cc + 1) * LANES)
                    rows[s4, r, sl] = jnp.maximum(
                        rows[s4, r, sl] + eawb[b, r, sl], 0.0)
                return carry2
            lax.fori_loop(0, ch, crow, 0)

            pltpu.async_copy(rows.at[s4], acc.at[ibuf.at[s8, 1]],
                             sems[b], add=True)
            if pf_g:
                pltpu.async_copy(eaw.at[pl.ds(boff + 2 * ch, ch)],
                                 eawb.at[b], seme[b])
            if pf_idx:
                issue_idx(j + 4, (s8 + 4) % 8)

        for j in range(4):
            issue_idx(j, j)
        for j in range(2):
            wait_idx(j, j)
            pltpu.async_copy(hx.at[ibuf.at[j, 0]], rows.at[j], semg[j])
            pltpu.async_copy(eaw.at[pl.ds(ebase + j * ch, ch)],
                             eawb.at[j], seme[j])

        tail_start = ((nchunk - 4) // 8) * 8
        for j in range(8):
            process(j, j, True, True, j >= 2)

        def step(g, carry):
            for b8 in range(8):
                process(8 * g + b8, b8, True, True, True)
            return carry
        lax.fori_loop(1, tail_start // 8, step, 0)
        for j in range(tail_start, nchunk):
            process(j, j % 8, j + 4 < nchunk, j + 2 < nchunk, True)
        for j in (nchunk - 2, nchunk - 1):
            pltpu.make_async_copy(
                rows.at[j % 4], acc.at[ibuf.at[j % 8, 1]],
                sems[j % 2]).wait()

        plsc.subcore_barrier()
        pltpu.sync_copy(acc.at[pl.ds(base_row, rpt)],
                        out_s.at[c, pl.ds(base_row, rpt)])

    return pl.kernel(body, out_type=out_type, mesh=mesh,
                     scratch_types=scratch)


def _make_cnt_kernel(nn, ee, dd):
    w = NC * NS
    ept = ee // w
    ch = 40
    nchunk = ept // ch
    rpt, nnp = _padded_rows(nn)
    nz = rpt // ch
    assert ept % ch == 0 and rpt % ch == 0
    assert nchunk % 2 == 0 and nchunk >= 8

    mesh = plsc.VectorSubcoreMesh(core_axis_name="c", subcore_axis_name="s",
                                  num_cores=NC, num_subcores=NS)
    out_type = jax.ShapeDtypeStruct((NC, nnp, dd), jnp.float32)
    scratch = (
        [pltpu.VMEM((8, 1, ch), jnp.int32),
         pltpu.VMEM((ch, dd), jnp.float32),
         pltpu.VMEM((ch, dd), jnp.float32),
         pltpu.VMEM_SHARED((nnp, dd), jnp.float32)]
        + [pltpu.SemaphoreType.DMA] * 6
    )

    def body(dstr, out_c, dsti, ones, zbuf, acccnt, *sem):
        c = lax.axis_index("c")
        s = lax.axis_index("s")
        semi = list(sem[0:4])
        sems = list(sem[4:6])

        def fill(r, carry):
            for cc in range(dd // LANES):
                sl = slice(cc * LANES, (cc + 1) * LANES)
                ones[r, sl] = jnp.ones((LANES,), jnp.float32)
                zbuf[r, sl] = jnp.zeros((LANES,), jnp.float32)
            return carry
        lax.fori_loop(0, ch, fill, 0)

        base_row = s * rpt
        for z in range(nz):
            pltpu.sync_copy(zbuf, acccnt.at[pl.ds(base_row + z * ch, ch)])

        plsc.subcore_barrier()

        ebase = (c * NS + s) * ept

        def issue_idx(j, s8):
            pltpu.async_copy(dstr.at[pl.ds(ebase + j * ch, ch)],
                             dsti.at[s8, 0], semi[s8 % 4])

        def process(j, s8, pf_idx, drain):
            b = s8 % 2
            pltpu.make_async_copy(dstr.at[pl.ds(ebase + j * ch, ch)],
                                  dsti.at[s8, 0], semi[s8 % 4]).wait()
            if drain:
                pltpu.make_async_copy(ones, acccnt.at[dsti.at[(s8 + 6) % 8, 0]],
                                      sems[b]).wait()
            pltpu.async_copy(ones, acccnt.at[dsti.at[s8, 0]], sems[b],
                             add=True)
            if pf_idx:
                issue_idx(j + 4, (s8 + 4) % 8)

        for j in range(4):
            issue_idx(j, j)

        tail_start = ((nchunk - 4) // 8) * 8
        for j in range(8):
            process(j, j, True, j >= 2)

        def step(g, carry):
            for b8 in range(8):
                process(8 * g + b8, b8, True, True)
            return carry
        lax.fori_loop(1, tail_start // 8, step, 0)
        for j in range(tail_start, nchunk):
            process(j, j % 8, j + 4 < nchunk, True)
        for j in (nchunk - 2, nchunk - 1):
            pltpu.make_async_copy(ones, acccnt.at[dsti.at[j % 8, 0]],
                                  sems[j % 2]).wait()

        plsc.subcore_barrier()
        pltpu.sync_copy(acccnt.at[pl.ds(base_row, rpt)],
                        out_c.at[c, pl.ds(base_row, rpt)])

    return pl.kernel(body, out_type=out_type, mesh=mesh,
                     scratch_types=scratch)


def _make_edge_update_kernel(nn, dp, ee, e0, esz):
    w = NC * NS
    ept = esz // w
    ch = 40
    nchunk = ept // ch
    assert ept % ch == 0
    assert nchunk >= 8

    mesh = plsc.VectorSubcoreMesh(core_axis_name="c", subcore_axis_name="s",
                                  num_cores=NC, num_subcores=NS)
    out_type = jax.ShapeDtypeStruct((esz, dp), jnp.float32)
    scratch = (
        [pltpu.VMEM((8, 2, ch), jnp.int32),
         pltpu.VMEM((4, ch, dp), jnp.float32),
         pltpu.VMEM((4, ch, dp), jnp.float32),
         pltpu.VMEM((4, ch, dp), jnp.float32)]
        + [pltpu.SemaphoreType.DMA] * 12
    )

    def body(hij, eaw2, srcr, dstr, out, ibuf, g1, g2, eb, *sem):
        c = lax.axis_index("c")
        s = lax.axis_index("s")
        semi = list(sem[0:4])
        sem1 = list(sem[4:6])
        sem2 = list(sem[6:8])
        seme = list(sem[8:10])
        semo = list(sem[10:12])
        ebase = (c * NS + s) * ept
        gbase = e0 + ebase

        def issue_idx(j, s8):
            boff = gbase + j * ch
            pltpu.async_copy(srcr.at[pl.ds(boff, ch)], ibuf.at[s8, 0],
                             semi[s8 % 4])
            pltpu.async_copy(dstr.at[pl.ds(boff, ch)], ibuf.at[s8, 1],
                             semi[s8 % 4])

        def wait_idx(j, s8):
            boff = gbase + j * ch
            pltpu.make_async_copy(srcr.at[pl.ds(boff, ch)],
                                  ibuf.at[s8, 0], semi[s8 % 4]).wait()
            pltpu.make_async_copy(dstr.at[pl.ds(boff, ch)],
                                  ibuf.at[s8, 1], semi[s8 % 4]).wait()

        def issue_loads(j, s8):
            boff = gbase + j * ch
            s4 = s8 % 4
            pltpu.async_copy(hij.at[ibuf.at[s8, 0]], g1.at[s4],
                             sem1[s8 % 2])
            pltpu.async_copy(hij.at[ibuf.at[s8, 1]], g2.at[s4],
                             sem2[s8 % 2])
            pltpu.async_copy(eaw2.at[pl.ds(boff, ch)], eb.at[s4],
                             seme[s8 % 2])

        def process(j, s8, pf_idx, pf_g, drain):
            b = s8 % 2
            s4 = s8 % 4
            boff = ebase + j * ch
            goff = gbase + j * ch
            pltpu.make_async_copy(hij.at[ibuf.at[s8, 0]], g1.at[s4],
                                  sem1[b]).wait()
            pltpu.make_async_copy(hij.at[ibuf.at[s8, 1]], g2.at[s4],
                                  sem2[b]).wait()
            pltpu.make_async_copy(eaw2.at[pl.ds(goff, ch)], eb.at[s4],
                                  seme[b]).wait()

            def crow(r, carry2):
                v = (eb[s4, r, 0:LANES] + g1[s4, r, 0:LANES]
                     + g2[s4, r, LANES:2 * LANES])
                eb[s4, r, 0:LANES] = jnp.maximum(v, 0.0)
                return carry2
            lax.fori_loop(0, ch, crow, 0)

            if drain:
                pltpu.make_async_copy(
                    eb.at[(s4 + 2) % 4],
                    out.at[pl.ds(boff - 2 * ch, ch)], semo[b]).wait()
            pltpu.async_copy(eb.at[s4], out.at[pl.ds(boff, ch)], semo[b])
            if pf_idx:
                issue_idx(j + 4, (s8 + 4) % 8)
            if pf_g:
                wait_idx(j + 2, (s8 + 2) % 8)
                issue_loads(j + 2, (s8 + 2) % 8)

        for j in range(4):
            issue_idx(j, j)
        for j in range(2):
            wait_idx(j, j)
            issue_loads(j, j)

        tail_start = ((nchunk - 4) // 8) * 8
        for j in range(8):
            process(j, j, True, True, j >= 2)

        def step(g, carry):
            for b8 in range(8):
                process(8 * g + b8, b8, True, True, True)
            return carry
        lax.fori_loop(1, tail_start // 8, step, 0)
        for j in range(tail_start, nchunk):
            process(j, j % 8, j + 4 < nchunk, j + 2 < nchunk, True)
        for j in (nchunk - 2, nchunk - 1):
            pltpu.make_async_copy(
                eb.at[j % 4], out.at[pl.ds(ebase + j * ch, ch)],
                semo[j % 2]).wait()

    return pl.kernel(body, out_type=out_type, mesh=mesh,
                     scratch_types=scratch)


def kernel(x, edge_attr, edge_index, Wm, bm, Wa, ba, We, be):
    n, d = x.shape
    e, de = edge_attr.shape
    nl = Wm.shape[0]
    assert de == LANES

    src = edge_index[0].astype(jnp.int32)
    dst = edge_index[1].astype(jnp.int32)

    bn = 400
    be_blk = 2000

    dp = 128

    msg = _make_msg_kernel(n, d, e)
    cntk = _make_cnt_kernel(n, e, d)
    edge_upd = _make_edge_update_kernel(n, dp, e, 0, e)

    h = x
    ea = edge_attr
    hx = _tc_node_matmul(x, Wm[0][:d], bn)
    cp = cntk(dst)
    for l in range(nl):
        last = l == nl - 1
        eaw = _edge_prep(ea, de, Wm[l][d:], bm[l][None], be_blk)
        if not last:
            wee_p = jnp.pad(We[l][2 * d:], ((0, 0), (0, dp - de)))
            bev_p = jnp.pad(be[l], (0, dp - de))[None]
            eaw2 = _edge_prep2(ea, de, wee_p, bev_p, be_blk)
        sp = msg(hx, eaw, src, dst)
        if not last:
            wij_p = jnp.pad(
                jnp.concatenate([We[l][:d], We[l][d:2 * d]], axis=1),
                ((0, 0), (0, dp - 2 * de)))
            h, hx, hij = _update(sp, cp, h, Wa[l][:d], Wa[l][d:],
                                 ba[l][None], Wm[l + 1][:d], wij_p, bn)
            ea = edge_upd(hij, eaw2, src, dst)
        else:
            h = _update(sp, cp, h, Wa[l][:d], Wa[l][d:], ba[l][None],
                        None, None, bn)
    return h


# speedup vs baseline: 1.0321x; 1.0148x over previous
import functools

import jax
import jax.numpy as jnp
from jax import lax
from jax.experimental import pallas as pl
from jax.experimental.pallas import tpu as pltpu
from jax.experimental.pallas import tpu_sc as plsc

NC = 2
NS = 16
LANES = 16


def _prep0_body(x_ref, w_ref, o_ref):
    o_ref[...] = jnp.dot(x_ref[...], w_ref[...],
                         preferred_element_type=jnp.float32)


def _tc_node_matmul(x, w, bn):
    n, d = x.shape
    return pl.pallas_call(
        _prep0_body,
        grid=(n // bn,),
        in_specs=[
            pl.BlockSpec((bn, d), lambda i: (i, 0)),
            pl.BlockSpec((d, w.shape[1]), lambda i: (0, 0)),
        ],
        out_specs=pl.BlockSpec((bn, w.shape[1]), lambda i: (i, 0)),
        out_shape=jax.ShapeDtypeStruct((n, w.shape[1]), jnp.float32),
    )(x, w)


def _edge_prep2_body(de, ea_ref, wee_ref, be_ref, eaw2_ref):
    ea = ea_ref[...][:, 0:de]
    eaw2_ref[...] = jnp.dot(ea, wee_ref[...],
                            preferred_element_type=jnp.float32) + be_ref[...]


def _edge_prep1_body(de, ea_ref, wme_ref, bm_ref, eaw_ref):
    ea = ea_ref[...][:, 0:de]
    eaw_ref[...] = jnp.dot(ea, wme_ref[...],
                           preferred_element_type=jnp.float32) + bm_ref[...]


def _edge_prep(ea, de, wme, bmv, be_blk):
    e, din = ea.shape
    d = wme.shape[1]
    return pl.pallas_call(
        functools.partial(_edge_prep1_body, de),
        grid=(e // be_blk,),
        in_specs=[
            pl.BlockSpec((be_blk, din), lambda i: (i, 0)),
            pl.BlockSpec((de, d), lambda i: (0, 0)),
            pl.BlockSpec((1, d), lambda i: (0, 0)),
        ],
        out_specs=pl.BlockSpec((be_blk, d), lambda i: (i, 0)),
        out_shape=jax.ShapeDtypeStruct((e, d), jnp.float32),
    )(ea, wme, bmv)


def _edge_prep1_alias_body(de, ea_ref, wme_ref, bm_ref, buf_ref, eaw_ref):
    del buf_ref
    ea = ea_ref[...][:, 0:de]
    eaw_ref[...] = jnp.dot(ea, wme_ref[...],
                           preferred_element_type=jnp.float32) + bm_ref[...]


def _edge_prep_half(ea_half, de, wme, bmv, be_blk, e_total, half_idx,
                    eaw_buf=None):
    eh, din = ea_half.shape
    d = wme.shape[1]
    nb = eh // be_blk
    off = half_idx * nb
    if eaw_buf is None:
        return pl.pallas_call(
            functools.partial(_edge_prep1_body, de),
            grid=(nb,),
            in_specs=[
                pl.BlockSpec((be_blk, din), lambda i: (i, 0)),
                pl.BlockSpec((de, d), lambda i: (0, 0)),
                pl.BlockSpec((1, d), lambda i: (0, 0)),
            ],
            out_specs=pl.BlockSpec((be_blk, d), lambda i: (i + off, 0)),
            out_shape=jax.ShapeDtypeStruct((e_total, d), jnp.float32),
        )(ea_half, wme, bmv)
    return pl.pallas_call(
        functools.partial(_edge_prep1_alias_body, de),
        grid=(nb,),
        in_specs=[
            pl.BlockSpec((be_blk, din), lambda i: (i, 0)),
            pl.BlockSpec((de, d), lambda i: (0, 0)),
            pl.BlockSpec((1, d), lambda i: (0, 0)),
            pl.BlockSpec((be_blk, d), lambda i: (i + off, 0)),
        ],
        out_specs=pl.BlockSpec((be_blk, d), lambda i: (i + off, 0)),
        out_shape=jax.ShapeDtypeStruct((e_total, d), jnp.float32),
        input_output_aliases={3: 0},
    )(ea_half, wme, bmv, eaw_buf)


def _edge_prep2_alias_body(de, ea_ref, wee_ref, be_ref, buf_ref, eaw2_ref):
    del buf_ref
    ea = ea_ref[...][:, 0:de]
    eaw2_ref[...] = jnp.dot(ea, wee_ref[...],
                            preferred_element_type=jnp.float32) + be_ref[...]


def _edge_prep2_half(ea_half, de, wee_p, bev_p, be_blk, e_total, half_idx,
                     buf=None):
    eh, din = ea_half.shape
    dp = wee_p.shape[1]
    nb = eh // be_blk
    off = half_idx * nb
    if buf is None:
        return pl.pallas_call(
            functools.partial(_edge_prep2_body, de),
            grid=(nb,),
            in_specs=[
                pl.BlockSpec((be_blk, din), lambda i: (i, 0)),
                pl.BlockSpec((de, dp), lambda i: (0, 0)),
                pl.BlockSpec((1, dp), lambda i: (0, 0)),
            ],
            out_specs=pl.BlockSpec((be_blk, dp), lambda i: (i + off, 0)),
            out_shape=jax.ShapeDtypeStruct((e_total, dp), jnp.float32),
        )(ea_half, wee_p, bev_p)
    return pl.pallas_call(
        functools.partial(_edge_prep2_alias_body, de),
        grid=(nb,),
        in_specs=[
            pl.BlockSpec((be_blk, din), lambda i: (i, 0)),
            pl.BlockSpec((de, dp), lambda i: (0, 0)),
            pl.BlockSpec((1, dp), lambda i: (0, 0)),
            pl.BlockSpec((be_blk, dp), lambda i: (i + off, 0)),
        ],
        out_specs=pl.BlockSpec((be_blk, dp), lambda i: (i + off, 0)),
        out_shape=jax.ShapeDtypeStruct((e_total, dp), jnp.float32),
        input_output_aliases={3: 0},
    )(ea_half, wee_p, bev_p, buf)


def _edge_prep2(ea, de, wee_p, bev_p, be_blk):
    e, din = ea.shape
    dp = wee_p.shape[1]
    return pl.pallas_call(
        functools.partial(_edge_prep2_body, de),
        grid=(e // be_blk,),
        in_specs=[
            pl.BlockSpec((be_blk, din), lambda i: (i, 0)),
            pl.BlockSpec((de, dp), lambda i: (0, 0)),
            pl.BlockSpec((1, dp), lambda i: (0, 0)),
        ],
        out_specs=pl.BlockSpec((be_blk, dp), lambda i: (i, 0)),
        out_shape=jax.ShapeDtypeStruct((e, dp), jnp.float32),
    )(ea, wee_p, bev_p)


def _update2_body(sp_ref, cp_ref, h_ref, waa_ref, wah_ref, ba_ref,
                  wmxn_ref, wij_ref,
                  hn_ref, hxn_ref, hij_ref):
    s = sp_ref[0] + sp_ref[1]
    cnt = cp_ref[0, :, 0:1] + cp_ref[1, :, 0:1]
    agg = s * (1.0 / jnp.maximum(cnt, 1.0))
    u = jnp.dot(agg, waa_ref[...], preferred_element_type=jnp.float32)
    u = u + jnp.dot(h_ref[...], wah_ref[...],
                    preferred_element_type=jnp.float32)
    u = jnp.maximum(u + ba_ref[...], 0.0)
    nn = jnp.sqrt(jnp.sum(u * u, axis=1, keepdims=True))
    hv = u / jnp.maximum(nn, 1e-12)
    hn_ref[...] = hv
    hxn_ref[...] = jnp.dot(hv, wmxn_ref[...],
                           preferred_element_type=jnp.float32)
    hij_ref[...] = jnp.dot(hv, wij_ref[...],
                           preferred_element_type=jnp.float32)


def _update1_body(sp_ref, cp_ref, h_ref, waa_ref, wah_ref, ba_ref, hn_ref):
    s = sp_ref[0] + sp_ref[1]
    cnt = cp_ref[0, :, 0:1] + cp_ref[1, :, 0:1]
    agg = s * (1.0 / jnp.maximum(cnt, 1.0))
    u = jnp.dot(agg, waa_ref[...], preferred_element_type=jnp.float32)
    u = u + jnp.dot(h_ref[...], wah_ref[...],
                    preferred_element_type=jnp.float32)
    u = jnp.maximum(u + ba_ref[...], 0.0)
    nn = jnp.sqrt(jnp.sum(u * u, axis=1, keepdims=True))
    hn_ref[...] = u / jnp.maximum(nn, 1e-12)


def _update(sp, cp, h, waa, wah, bav, wmxn, wij_p, bn):
    n, d = h.shape
    de = cp.shape[2]
    grid = (n // bn,)
    common_in = [
        pl.BlockSpec((NC, bn, d), lambda i: (0, i, 0)),
        pl.BlockSpec((NC, bn, de), lambda i: (0, i, 0)),
        pl.BlockSpec((bn, d), lambda i: (i, 0)),
        pl.BlockSpec((d, d), lambda i: (0, 0)),
        pl.BlockSpec((d, d), lambda i: (0, 0)),
        pl.BlockSpec((1, d), lambda i: (0, 0)),
    ]
    if wmxn is None:
        return pl.pallas_call(
            _update1_body,
            grid=grid,
            in_specs=common_in,
            out_specs=pl.BlockSpec((bn, d), lambda i: (i, 0)),
            out_shape=jax.ShapeDtypeStruct((n, d), jnp.float32),
        )(sp, cp, h, waa, wah, bav)
    dp = wij_p.shape[1]
    return pl.pallas_call(
        _update2_body,
        grid=grid,
        in_specs=common_in + [
            pl.BlockSpec((d, d), lambda i: (0, 0)),
            pl.BlockSpec((d, dp), lambda i: (0, 0)),
        ],
        out_specs=[
            pl.BlockSpec((bn, d), lambda i: (i, 0)),
            pl.BlockSpec((bn, d), lambda i: (i, 0)),
            pl.BlockSpec((bn, dp), lambda i: (i, 0)),
        ],
        out_shape=[
            jax.ShapeDtypeStruct((n, d), jnp.float32),
            jax.ShapeDtypeStruct((n, d), jnp.float32),
            jax.ShapeDtypeStruct((n, dp), jnp.float32),
        ],
    )(sp, cp, h, waa, wah, bav, wmxn, wij_p)


def _padded_rows(nn):
    rpt = -(-nn // NS)
    rpt = -(-rpt // 128) * 128
    return rpt, rpt * NS


def _make_msg_kernel(nn, dd, ee):
    w = NC * NS
    ept = ee // w
    ch = 40
    nchunk = ept // ch
    rpt, nnp = _padded_rows(nn)
    nz = rpt // ch
    assert ept % ch == 0 and rpt % ch == 0 and dd % LANES == 0
    assert nchunk % 2 == 0 and nchunk >= 8

    mesh = plsc.VectorSubcoreMesh(core_axis_name="c", subcore_axis_name="s",
                                  num_cores=NC, num_subcores=NS)

    out_type = jax.ShapeDtypeStruct((NC, nnp, dd), jnp.float32)
    scratch = (
        [pltpu.VMEM((8, 2, ch), jnp.int32),
         pltpu.VMEM((4, ch, dd), jnp.float32),
         pltpu.VMEM((2, ch, dd), jnp.float32),
         pltpu.VMEM_SHARED((nnp, dd), jnp.float32)]
        + [pltpu.SemaphoreType.DMA] * 10
    )

    def body(hx, eaw, srcr, dstr, out_s, ibuf, rows, eawb, acc, *sem):
        c = lax.axis_index("c")
        s = lax.axis_index("s")
        ncol = dd // LANES
        semi = list(sem[0:4])
        semg = list(sem[4:6])
        seme = list(sem[6:8])
        sems = list(sem[8:10])

        def zrow(r, carry):
            for cc in range(ncol):
                rows[0, r, cc * LANES:(cc + 1) * LANES] = jnp.zeros(
                    (LANES,), jnp.float32)
            return carry
        lax.fori_loop(0, ch, zrow, 0)

        base_row = s * rpt
        for z in range(nz):
            pltpu.sync_copy(rows.at[0], acc.at[pl.ds(base_row + z * ch, ch)])

        plsc.subcore_barrier()

        ebase = (c * NS + s) * ept

        def issue_idx(j, s8):
            boff = ebase + j * ch
            pltpu.async_copy(srcr.at[pl.ds(boff, ch)], ibuf.at[s8, 0],
                             semi[s8 % 4])
            pltpu.async_copy(dstr.at[pl.ds(boff, ch)], ibuf.at[s8, 1],
                             semi[s8 % 4])

        def wait_idx(j, s8):
            boff = ebase + j * ch
            pltpu.make_async_copy(srcr.at[pl.ds(boff, ch)], ibuf.at[s8, 0],
                                  semi[s8 % 4]).wait()
            pltpu.make_async_copy(dstr.at[pl.ds(boff, ch)], ibuf.at[s8, 1],
                                  semi[s8 % 4]).wait()

        def process(j, s8, pf_idx, pf_g, drain):
            b = s8 % 2
            s4 = s8 % 4
            boff = ebase + j * ch
            pltpu.make_async_copy(hx.at[ibuf.at[s8, 0]], rows.at[s4],
                                  semg[b]).wait()
            pltpu.make_async_copy(eaw.at[pl.ds(boff, ch)], eawb.at[b],
                                  seme[b]).wait()
            if drain:
                pltpu.make_async_copy(
                    rows.at[(s4 + 2) % 4], acc.at[ibuf.at[(s8 + 6) % 8, 1]],
                    sems[b]).wait()
            if pf_g:
                wait_idx(j + 2, (s8 + 2) % 8)
                pltpu.async_copy(hx.at[ibuf.at[(s8 + 2) % 8, 0]],
                                 rows.at[(s4 + 2) % 4], semg[b])

            def crow(r, carry2):
                for cc in range(ncol):
                    sl = slice(cc * LANES, (cc + 1) * LANES)
                    rows[s4, r, sl] = jnp.maximum(
                        rows[s4, r, sl] + eawb[b, r, sl], 0.0)
                return carry2
            lax.fori_loop(0, ch, crow, 0)

            pltpu.async_copy(rows.at[s4], acc.at[ibuf.at[s8, 1]],
                             sems[b], add=True)
            if pf_g:
                pltpu.async_copy(eaw.at[pl.ds(boff + 2 * ch, ch)],
                                 eawb.at[b], seme[b])
            if pf_idx:
                issue_idx(j + 4, (s8 + 4) % 8)

        for j in range(4):
            issue_idx(j, j)
        for j in range(2):
            wait_idx(j, j)
            pltpu.async_copy(hx.at[ibuf.at[j, 0]], rows.at[j], semg[j])
            pltpu.async_copy(eaw.at[pl.ds(ebase + j * ch, ch)],
                             eawb.at[j], seme[j])

        tail_start = ((nchunk - 4) // 8) * 8
        for j in range(8):
            process(j, j, True, True, j >= 2)

        def step(g, carry):
            for b8 in range(8):
                process(8 * g + b8, b8, True, True, True)
            return carry
        lax.fori_loop(1, tail_start // 8, step, 0)
        for j in range(tail_start, nchunk):
            process(j, j % 8, j + 4 < nchunk, j + 2 < nchunk, True)
        for j in (nchunk - 2, nchunk - 1):
            pltpu.make_async_copy(
                rows.at[j % 4], acc.at[ibuf.at[j % 8, 1]],
                sems[j % 2]).wait()

        plsc.subcore_barrier()
        pltpu.sync_copy(acc.at[pl.ds(base_row, rpt)],
                        out_s.at[c, pl.ds(base_row, rpt)])

    return pl.kernel(body, out_type=out_type, mesh=mesh,
                     scratch_types=scratch)


def _make_cnt_kernel(nn, ee, dd):
    w = NC * NS
    ept = ee // w
    ch = 40
    nchunk = ept // ch
    rpt, nnp = _padded_rows(nn)
    nz = rpt // ch
    assert ept % ch == 0 and rpt % ch == 0
    assert nchunk % 2 == 0 and nchunk >= 8

    mesh = plsc.VectorSubcoreMesh(core_axis_name="c", subcore_axis_name="s",
                                  num_cores=NC, num_subcores=NS)
    out_type = jax.ShapeDtypeStruct((NC, nnp, dd), jnp.float32)
    scratch = (
        [pltpu.VMEM((8, 1, ch), jnp.int32),
         pltpu.VMEM((ch, dd), jnp.float32),
         pltpu.VMEM((ch, dd), jnp.float32),
         pltpu.VMEM_SHARED((nnp, dd), jnp.float32)]
        + [pltpu.SemaphoreType.DMA] * 6
    )

    def body(dstr, out_c, dsti, ones, zbuf, acccnt, *sem):
        c = lax.axis_index("c")
        s = lax.axis_index("s")
        semi = list(sem[0:4])
        sems = list(sem[4:6])

        def fill(r, carry):
            for cc in range(dd // LANES):
                sl = slice(cc * LANES, (cc + 1) * LANES)
                ones[r, sl] = jnp.ones((LANES,), jnp.float32)
                zbuf[r, sl] = jnp.zeros((LANES,), jnp.float32)
            return carry
        lax.fori_loop(0, ch, fill, 0)

        base_row = s * rpt
        for z in range(nz):
            pltpu.sync_copy(zbuf, acccnt.at[pl.ds(base_row + z * ch, ch)])

        plsc.subcore_barrier()

        ebase = (c * NS + s) * ept

        def issue_idx(j, s8):
            pltpu.async_copy(dstr.at[pl.ds(ebase + j * ch, ch)],
                             dsti.at[s8, 0], semi[s8 % 4])

        def process(j, s8, pf_idx, drain):
            b = s8 % 2
            pltpu.make_async_copy(dstr.at[pl.ds(ebase + j * ch, ch)],
                                  dsti.at[s8, 0], semi[s8 % 4]).wait()
            if drain:
                pltpu.make_async_copy(ones, acccnt.at[dsti.at[(s8 + 6) % 8, 0]],
                                      sems[b]).wait()
            pltpu.async_copy(ones, acccnt.at[dsti.at[s8, 0]], sems[b],
                             add=True)
            if pf_idx:
                issue_idx(j + 4, (s8 + 4) % 8)

        for j in range(4):
            issue_idx(j, j)

        tail_start = ((nchunk - 4) // 8) * 8
        for j in range(8):
            process(j, j, True, j >= 2)

        def step(g, carry):
            for b8 in range(8):
                process(8 * g + b8, b8, True, True)
            return carry
        lax.fori_loop(1, tail_start // 8, step, 0)
        for j in range(tail_start, nchunk):
            process(j, j % 8, j + 4 < nchunk, True)
        for j in (nchunk - 2, nchunk - 1):
            pltpu.make_async_copy(ones, acccnt.at[dsti.at[j % 8, 0]],
                                  sems[j % 2]).wait()

        plsc.subcore_barrier()
        pltpu.sync_copy(acccnt.at[pl.ds(base_row, rpt)],
                        out_c.at[c, pl.ds(base_row, rpt)])

    return pl.kernel(body, out_type=out_type, mesh=mesh,
                     scratch_types=scratch)


def _make_edge_update_kernel(nn, dp, ee, e0, esz):
    w = NC * NS
    ept = esz // w
    ch = 40
    nchunk = ept // ch
    assert ept % ch == 0
    assert nchunk >= 8

    mesh = plsc.VectorSubcoreMesh(core_axis_name="c", subcore_axis_name="s",
                                  num_cores=NC, num_subcores=NS)
    out_type = jax.ShapeDtypeStruct((esz, dp), jnp.float32)
    scratch = (
        [pltpu.VMEM((8, 2, ch), jnp.int32),
         pltpu.VMEM((4, ch, dp), jnp.float32),
         pltpu.VMEM((4, ch, dp), jnp.float32),
         pltpu.VMEM((4, ch, dp), jnp.float32)]
        + [pltpu.SemaphoreType.DMA] * 12
    )

    def body(hij, eaw2, srcr, dstr, out, ibuf, g1, g2, eb, *sem):
        c = lax.axis_index("c")
        s = lax.axis_index("s")
        semi = list(sem[0:4])
        sem1 = list(sem[4:6])
        sem2 = list(sem[6:8])
        seme = list(sem[8:10])
        semo = list(sem[10:12])
        ebase = (c * NS + s) * ept
        gbase = e0 + ebase

        def issue_idx(j, s8):
            boff = gbase + j * ch
            pltpu.async_copy(srcr.at[pl.ds(boff, ch)], ibuf.at[s8, 0],
                             semi[s8 % 4])
            pltpu.async_copy(dstr.at[pl.ds(boff, ch)], ibuf.at[s8, 1],
                             semi[s8 % 4])

        def wait_idx(j, s8):
            boff = gbase + j * ch
            pltpu.make_async_copy(srcr.at[pl.ds(boff, ch)],
                                  ibuf.at[s8, 0], semi[s8 % 4]).wait()
            pltpu.make_async_copy(dstr.at[pl.ds(boff, ch)],
                                  ibuf.at[s8, 1], semi[s8 % 4]).wait()

        def issue_loads(j, s8):
            boff = gbase + j * ch
            s4 = s8 % 4
            pltpu.async_copy(hij.at[ibuf.at[s8, 0]], g1.at[s4],
                             sem1[s8 % 2])
            pltpu.async_copy(hij.at[ibuf.at[s8, 1]], g2.at[s4],
                             sem2[s8 % 2])
            pltpu.async_copy(eaw2.at[pl.ds(boff, ch)], eb.at[s4],
                             seme[s8 % 2])

        def process(j, s8, pf_idx, pf_g, drain):
            b = s8 % 2
            s4 = s8 % 4
            boff = ebase + j * ch
            goff = gbase + j * ch
            pltpu.make_async_copy(hij.at[ibuf.at[s8, 0]], g1.at[s4],
                                  sem1[b]).wait()
            pltpu.make_async_copy(hij.at[ibuf.at[s8, 1]], g2.at[s4],
                                  sem2[b]).wait()
            pltpu.make_async_copy(eaw2.at[pl.ds(goff, ch)], eb.at[s4],
                                  seme[b]).wait()
            if drain:
                pltpu.make_async_copy(
                    eb.at[(s4 + 2) % 4],
                    out.at[pl.ds(boff - 2 * ch, ch)], semo[b]).wait()
            if pf_g:
                wait_idx(j + 2, (s8 + 2) % 8)
                issue_loads(j + 2, (s8 + 2) % 8)

            def crow(r, carry2):
                v = (eb[s4, r, 0:LANES] + g1[s4, r, 0:LANES]
                     + g2[s4, r, LANES:2 * LANES])
                eb[s4, r, 0:LANES] = jnp.maximum(v, 0.0)
                return carry2
            lax.fori_loop(0, ch, crow, 0)

            pltpu.async_copy(eb.at[s4], out.at[pl.ds(boff, ch)], semo[b])
            if pf_idx:
                issue_idx(j + 4, (s8 + 4) % 8)

        for j in range(4):
            issue_idx(j, j)
        for j in range(2):
            wait_idx(j, j)
            issue_loads(j, j)

        tail_start = ((nchunk - 4) // 8) * 8
        for j in range(8):
            process(j, j, True, True, j >= 2)

        def step(g, carry):
            for b8 in range(8):
                process(8 * g + b8, b8, True, True, True)
            return carry
        lax.fori_loop(1, tail_start // 8, step, 0)
        for j in range(tail_start, nchunk):
            process(j, j % 8, j + 4 < nchunk, j + 2 < nchunk, True)
        for j in (nchunk - 2, nchunk - 1):
            pltpu.make_async_copy(
                eb.at[j % 4], out.at[pl.ds(ebase + j * ch, ch)],
                semo[j % 2]).wait()

    return pl.kernel(body, out_type=out_type, mesh=mesh,
                     scratch_types=scratch)


def kernel(x, edge_attr, edge_index, Wm, bm, Wa, ba, We, be):
    n, d = x.shape
    e, de = edge_attr.shape
    nl = Wm.shape[0]
    assert de == LANES

    src = edge_index[0].astype(jnp.int32)
    dst = edge_index[1].astype(jnp.int32)

    bn = 400
    be_blk = 2000

    dp = 128

    msg = _make_msg_kernel(n, d, e)
    cntk = _make_cnt_kernel(n, e, d)
    edge_upd = _make_edge_update_kernel(n, dp, e, 0, e)

    h = x
    ea = edge_attr
    hx = _tc_node_matmul(x, Wm[0][:d], bn)
    cp = cntk(dst)
    for l in range(nl):
        last = l == nl - 1
        eaw = _edge_prep(ea, de, Wm[l][d:], bm[l][None], be_blk)
        if not last:
            wee_p = jnp.pad(We[l][2 * d:], ((0, 0), (0, dp - de)))
            bev_p = jnp.pad(be[l], (0, dp - de))[None]
            eaw2 = _edge_prep2(ea, de, wee_p, bev_p, be_blk)
        sp = msg(hx, eaw, src, dst)
        if not last:
            wij_p = jnp.pad(
                jnp.concatenate([We[l][:d], We[l][d:2 * d]], axis=1),
                ((0, 0), (0, dp - 2 * de)))
            h, hx, hij = _update(sp, cp, h, Wa[l][:d], Wa[l][d:],
                                 ba[l][None], Wm[l + 1][:d], wij_p, bn)
            ea = edge_upd(hij, eaw2, src, dst)
        else:
            h = _update(sp, cp, h, Wa[l][:d], Wa[l][d:], ba[l][None],
                        None, None, bn)
    return h


# speedup vs baseline: 1.1061x; 1.0718x over previous
import functools

import jax
import jax.numpy as jnp
from jax import lax
from jax.experimental import pallas as pl
from jax.experimental.pallas import tpu as pltpu
from jax.experimental.pallas import tpu_sc as plsc

NC = 2
NS = 16
LANES = 16


def _prep0_body(x_ref, w_ref, o_ref):
    o_ref[...] = jnp.dot(x_ref[...], w_ref[...],
                         preferred_element_type=jnp.float32)


def _tc_node_matmul(x, w, bn):
    n, d = x.shape
    return pl.pallas_call(
        _prep0_body,
        grid=(n // bn,),
        in_specs=[
            pl.BlockSpec((bn, d), lambda i: (i, 0)),
            pl.BlockSpec((d, w.shape[1]), lambda i: (0, 0)),
        ],
        out_specs=pl.BlockSpec((bn, w.shape[1]), lambda i: (i, 0)),
        out_shape=jax.ShapeDtypeStruct((n, w.shape[1]), jnp.float32),
    )(x, w)


def _edge_prep2_body(de, ea_ref, wee_ref, be_ref, eaw2_ref):
    ea = ea_ref[...][:, 0:de]
    eaw2_ref[...] = jnp.dot(ea, wee_ref[...],
                            preferred_element_type=jnp.float32) + be_ref[...]


def _edge_prep1_body(de, ea_ref, wme_ref, bm_ref, eaw_ref):
    ea = ea_ref[...][:, 0:de]
    eaw_ref[...] = jnp.dot(ea, wme_ref[...],
                           preferred_element_type=jnp.float32) + bm_ref[...]


def _edge_prep(ea, de, wme, bmv, be_blk):
    e, din = ea.shape
    d = wme.shape[1]
    return pl.pallas_call(
        functools.partial(_edge_prep1_body, de),
        grid=(e // be_blk,),
        in_specs=[
            pl.BlockSpec((be_blk, din), lambda i: (i, 0)),
            pl.BlockSpec((de, d), lambda i: (0, 0)),
            pl.BlockSpec((1, d), lambda i: (0, 0)),
        ],
        out_specs=pl.BlockSpec((be_blk, d), lambda i: (i, 0)),
        out_shape=jax.ShapeDtypeStruct((e, d), jnp.float32),
    )(ea, wme, bmv)


def _edge_prep1_alias_body(de, ea_ref, wme_ref, bm_ref, buf_ref, eaw_ref):
    del buf_ref
    ea = ea_ref[...][:, 0:de]
    eaw_ref[...] = jnp.dot(ea, wme_ref[...],
                           preferred_element_type=jnp.float32) + bm_ref[...]


def _edge_prep_half(ea_half, de, wme, bmv, be_blk, e_total, half_idx,
                    eaw_buf=None):
    eh, din = ea_half.shape
    d = wme.shape[1]
    nb = eh // be_blk
    off = half_idx * nb
    if eaw_buf is None:
        return pl.pallas_call(
            functools.partial(_edge_prep1_body, de),
            grid=(nb,),
            in_specs=[
                pl.BlockSpec((be_blk, din), lambda i: (i, 0)),
                pl.BlockSpec((de, d), lambda i: (0, 0)),
                pl.BlockSpec((1, d), lambda i: (0, 0)),
            ],
            out_specs=pl.BlockSpec((be_blk, d), lambda i: (i + off, 0)),
            out_shape=jax.ShapeDtypeStruct((e_total, d), jnp.float32),
        )(ea_half, wme, bmv)
    return pl.pallas_call(
        functools.partial(_edge_prep1_alias_body, de),
        grid=(nb,),
        in_specs=[
            pl.BlockSpec((be_blk, din), lambda i: (i, 0)),
            pl.BlockSpec((de, d), lambda i: (0, 0)),
            pl.BlockSpec((1, d), lambda i: (0, 0)),
            pl.BlockSpec((be_blk, d), lambda i: (i + off, 0)),
        ],
        out_specs=pl.BlockSpec((be_blk, d), lambda i: (i + off, 0)),
        out_shape=jax.ShapeDtypeStruct((e_total, d), jnp.float32),
        input_output_aliases={3: 0},
    )(ea_half, wme, bmv, eaw_buf)


def _edge_prep2_alias_body(de, ea_ref, wee_ref, be_ref, buf_ref, eaw2_ref):
    del buf_ref
    ea = ea_ref[...][:, 0:de]
    eaw2_ref[...] = jnp.dot(ea, wee_ref[...],
                            preferred_element_type=jnp.float32) + be_ref[...]


def _edge_prep2_half(ea_half, de, wee_p, bev_p, be_blk, e_total, half_idx,
                     buf=None):
    eh, din = ea_half.shape
    dp = wee_p.shape[1]
    nb = eh // be_blk
    off = half_idx * nb
    if buf is None:
        return pl.pallas_call(
            functools.partial(_edge_prep2_body, de),
            grid=(nb,),
            in_specs=[
                pl.BlockSpec((be_blk, din), lambda i: (i, 0)),
                pl.BlockSpec((de, dp), lambda i: (0, 0)),
                pl.BlockSpec((1, dp), lambda i: (0, 0)),
            ],
            out_specs=pl.BlockSpec((be_blk, dp), lambda i: (i + off, 0)),
            out_shape=jax.ShapeDtypeStruct((e_total, dp), jnp.float32),
        )(ea_half, wee_p, bev_p)
    return pl.pallas_call(
        functools.partial(_edge_prep2_alias_body, de),
        grid=(nb,),
        in_specs=[
            pl.BlockSpec((be_blk, din), lambda i: (i, 0)),
            pl.BlockSpec((de, dp), lambda i: (0, 0)),
            pl.BlockSpec((1, dp), lambda i: (0, 0)),
            pl.BlockSpec((be_blk, dp), lambda i: (i + off, 0)),
        ],
        out_specs=pl.BlockSpec((be_blk, dp), lambda i: (i + off, 0)),
        out_shape=jax.ShapeDtypeStruct((e_total, dp), jnp.float32),
        input_output_aliases={3: 0},
    )(ea_half, wee_p, bev_p, buf)


def _edge_prep2(ea, de, wee_p, bev_p, be_blk):
    e, din = ea.shape
    dp = wee_p.shape[1]
    return pl.pallas_call(
        functools.partial(_edge_prep2_body, de),
        grid=(e // be_blk,),
        in_specs=[
            pl.BlockSpec((be_blk, din), lambda i: (i, 0)),
            pl.BlockSpec((de, dp), lambda i: (0, 0)),
            pl.BlockSpec((1, dp), lambda i: (0, 0)),
        ],
        out_specs=pl.BlockSpec((be_blk, dp), lambda i: (i, 0)),
        out_shape=jax.ShapeDtypeStruct((e, dp), jnp.float32),
    )(ea, wee_p, bev_p)


def _update2_body(sp_ref, cp_ref, h_ref, waa_ref, wah_ref, ba_ref,
                  wmxn_ref, wij_ref,
                  hn_ref, hxn_ref, hij_ref):
    s = sp_ref[0] + sp_ref[1]
    cnt = cp_ref[0, :, 0:1] + cp_ref[1, :, 0:1]
    agg = s * (1.0 / jnp.maximum(cnt, 1.0))
    u = jnp.dot(agg, waa_ref[...], preferred_element_type=jnp.float32)
    u = u + jnp.dot(h_ref[...], wah_ref[...],
                    preferred_element_type=jnp.float32)
    u = jnp.maximum(u + ba_ref[...], 0.0)
    nn = jnp.sqrt(jnp.sum(u * u, axis=1, keepdims=True))
    hv = u / jnp.maximum(nn, 1e-12)
    hn_ref[...] = hv
    hxn_ref[...] = jnp.dot(hv, wmxn_ref[...],
                           preferred_element_type=jnp.float32)
    hij_ref[...] = jnp.dot(hv, wij_ref[...],
                           preferred_element_type=jnp.float32)


def _update1_body(sp_ref, cp_ref, h_ref, waa_ref, wah_ref, ba_ref, hn_ref):
    s = sp_ref[0] + sp_ref[1]
    cnt = cp_ref[0, :, 0:1] + cp_ref[1, :, 0:1]
    agg = s * (1.0 / jnp.maximum(cnt, 1.0))
    u = jnp.dot(agg, waa_ref[...], preferred_element_type=jnp.float32)
    u = u + jnp.dot(h_ref[...], wah_ref[...],
                    preferred_element_type=jnp.float32)
    u = jnp.maximum(u + ba_ref[...], 0.0)
    nn = jnp.sqrt(jnp.sum(u * u, axis=1, keepdims=True))
    hn_ref[...] = u / jnp.maximum(nn, 1e-12)


def _update(sp, cp, h, waa, wah, bav, wmxn, wij_p, bn):
    n, d = h.shape
    de = cp.shape[2]
    grid = (n // bn,)
    common_in = [
        pl.BlockSpec((NC, bn, d), lambda i: (0, i, 0)),
        pl.BlockSpec((NC, bn, de), lambda i: (0, i, 0)),
        pl.BlockSpec((bn, d), lambda i: (i, 0)),
        pl.BlockSpec((d, d), lambda i: (0, 0)),
        pl.BlockSpec((d, d), lambda i: (0, 0)),
        pl.BlockSpec((1, d), lambda i: (0, 0)),
    ]
    if wmxn is None:
        return pl.pallas_call(
            _update1_body,
            grid=grid,
            in_specs=common_in,
            out_specs=pl.BlockSpec((bn, d), lambda i: (i, 0)),
            out_shape=jax.ShapeDtypeStruct((n, d), jnp.float32),
        )(sp, cp, h, waa, wah, bav)
    dp = wij_p.shape[1]
    return pl.pallas_call(
        _update2_body,
        grid=grid,
        in_specs=common_in + [
            pl.BlockSpec((d, d), lambda i: (0, 0)),
            pl.BlockSpec((d, dp), lambda i: (0, 0)),
        ],
        out_specs=[
            pl.BlockSpec((bn, d), lambda i: (i, 0)),
            pl.BlockSpec((bn, d), lambda i: (i, 0)),
            pl.BlockSpec((bn, dp), lambda i: (i, 0)),
        ],
        out_shape=[
            jax.ShapeDtypeStruct((n, d), jnp.float32),
            jax.ShapeDtypeStruct((n, d), jnp.float32),
            jax.ShapeDtypeStruct((n, dp), jnp.float32),
        ],
    )(sp, cp, h, waa, wah, bav, wmxn, wij_p)


def _padded_rows(nn):
    rpt = -(-nn // NS)
    rpt = -(-rpt // 128) * 128
    return rpt, rpt * NS


def _make_msg_kernel(nn, dd, ee):
    w = NC * NS
    ept = ee // w
    ch = 40
    nchunk = ept // ch
    rpt, nnp = _padded_rows(nn)
    nz = rpt // ch
    assert ept % ch == 0 and rpt % ch == 0 and dd % LANES == 0
    assert nchunk % 2 == 0 and nchunk >= 8

    mesh = plsc.VectorSubcoreMesh(core_axis_name="c", subcore_axis_name="s",
                                  num_cores=NC, num_subcores=NS)

    out_type = jax.ShapeDtypeStruct((NC, nnp, dd), jnp.float32)
    scratch = (
        [pltpu.VMEM((8, 2, ch), jnp.int32),
         pltpu.VMEM((4, ch, dd), jnp.float32),
         pltpu.VMEM((2, ch, dd), jnp.float32),
         pltpu.VMEM_SHARED((nnp, dd), jnp.float32)]
        + [pltpu.SemaphoreType.DMA] * 10
    )

    def body(hx, eaw, srcr, dstr, out_s, ibuf, rows, eawb, acc, *sem):
        c = lax.axis_index("c")
        s = lax.axis_index("s")
        ncol = dd // LANES
        semi = list(sem[0:4])
        semg = list(sem[4:6])
        seme = list(sem[6:8])
        sems = list(sem[8:10])

        def zrow(r, carry):
            for cc in range(ncol):
                rows[0, r, cc * LANES:(cc + 1) * LANES] = jnp.zeros(
                    (LANES,), jnp.float32)
            return carry
        lax.fori_loop(0, ch, zrow, 0)

        base_row = s * rpt
        for z in range(nz):
            pltpu.sync_copy(rows.at[0], acc.at[pl.ds(base_row + z * ch, ch)])

        plsc.subcore_barrier()

        ebase = (c * NS + s) * ept

        def issue_idx(j, s8):
            boff = ebase + j * ch
            pltpu.async_copy(srcr.at[pl.ds(boff, ch)], ibuf.at[s8, 0],
                             semi[s8 % 4])
            pltpu.async_copy(dstr.at[pl.ds(boff, ch)], ibuf.at[s8, 1],
                             semi[s8 % 4])

        def wait_idx(j, s8):
            boff = ebase + j * ch
            pltpu.make_async_copy(srcr.at[pl.ds(boff, ch)], ibuf.at[s8, 0],
                                  semi[s8 % 4]).wait()
            pltpu.make_async_copy(dstr.at[pl.ds(boff, ch)], ibuf.at[s8, 1],
                                  semi[s8 % 4]).wait()

        def process(j, s8, pf_idx, pf_g, drain):
            b = s8 % 2
            s4 = s8 % 4
            boff = ebase + j * ch
            pltpu.make_async_copy(hx.at[ibuf.at[s8, 0]], rows.at[s4],
                                  semg[b]).wait()
            pltpu.make_async_copy(eaw.at[pl.ds(boff, ch)], eawb.at[b],
                                  seme[b]).wait()
            if drain:
                pltpu.make_async_copy(
                    rows.at[(s4 + 2) % 4], acc.at[ibuf.at[(s8 + 6) % 8, 1]],
                    sems[b]).wait()
            if pf_g:
                wait_idx(j + 2, (s8 + 2) % 8)
                pltpu.async_copy(hx.at[ibuf.at[(s8 + 2) % 8, 0]],
                                 rows.at[(s4 + 2) % 4], semg[b])

            def crow(r, carry2):
                for cc in range(ncol):
                    sl = slice(cc * LANES, (cc + 1) * LANES)
                    rows[s4, r, sl] = jnp.maximum(
                        rows[s4, r, sl] + eawb[b, r, sl], 0.0)
                return carry2
            lax.fori_loop(0, ch, crow, 0)

            pltpu.async_copy(rows.at[s4], acc.at[ibuf.at[s8, 1]],
                             sems[b], add=True)
            if pf_g:
                pltpu.async_copy(eaw.at[pl.ds(boff + 2 * ch, ch)],
                                 eawb.at[b], seme[b])
            if pf_idx:
                issue_idx(j + 4, (s8 + 4) % 8)

        for j in range(4):
            issue_idx(j, j)
        for j in range(2):
            wait_idx(j, j)
            pltpu.async_copy(hx.at[ibuf.at[j, 0]], rows.at[j], semg[j])
            pltpu.async_copy(eaw.at[pl.ds(ebase + j * ch, ch)],
                             eawb.at[j], seme[j])

        tail_start = ((nchunk - 4) // 8) * 8
        for j in range(8):
            process(j, j, True, True, j >= 2)

        def step(g, carry):
            for b8 in range(8):
                process(8 * g + b8, b8, True, True, True)
            return carry
        lax.fori_loop(1, tail_start // 8, step, 0)
        for j in range(tail_start, nchunk):
            process(j, j % 8, j + 4 < nchunk, j + 2 < nchunk, True)
        for j in (nchunk - 2, nchunk - 1):
            pltpu.make_async_copy(
                rows.at[j % 4], acc.at[ibuf.at[j % 8, 1]],
                sems[j % 2]).wait()

        plsc.subcore_barrier()
        pltpu.sync_copy(acc.at[pl.ds(base_row, rpt)],
                        out_s.at[c, pl.ds(base_row, rpt)])

    return pl.kernel(body, out_type=out_type, mesh=mesh,
                     scratch_types=scratch)


def _make_cnt_kernel(nn, ee, dd):
    w = NC * NS
    ept = ee // w
    ch = 40
    nchunk = ept // ch
    rpt, nnp = _padded_rows(nn)
    nz = rpt // ch
    assert ept % ch == 0 and rpt % ch == 0
    assert nchunk % 2 == 0 and nchunk >= 8

    mesh = plsc.VectorSubcoreMesh(core_axis_name="c", subcore_axis_name="s",
                                  num_cores=NC, num_subcores=NS)
    out_type = jax.ShapeDtypeStruct((NC, nnp, dd), jnp.float32)
    scratch = (
        [pltpu.VMEM((8, 1, ch), jnp.int32),
         pltpu.VMEM((ch, dd), jnp.float32),
         pltpu.VMEM((ch, dd), jnp.float32),
         pltpu.VMEM_SHARED((nnp, dd), jnp.float32)]
        + [pltpu.SemaphoreType.DMA] * 6
    )

    def body(dstr, out_c, dsti, ones, zbuf, acccnt, *sem):
        c = lax.axis_index("c")
        s = lax.axis_index("s")
        semi = list(sem[0:4])
        sems = list(sem[4:6])

        def fill(r, carry):
            for cc in range(dd // LANES):
                sl = slice(cc * LANES, (cc + 1) * LANES)
                ones[r, sl] = jnp.ones((LANES,), jnp.float32)
                zbuf[r, sl] = jnp.zeros((LANES,), jnp.float32)
            return carry
        lax.fori_loop(0, ch, fill, 0)

        base_row = s * rpt
        for z in range(nz):
            pltpu.sync_copy(zbuf, acccnt.at[pl.ds(base_row + z * ch, ch)])

        plsc.subcore_barrier()

        ebase = (c * NS + s) * ept

        def issue_idx(j, s8):
            pltpu.async_copy(dstr.at[pl.ds(ebase + j * ch, ch)],
                             dsti.at[s8, 0], semi[s8 % 4])

        def process(j, s8, pf_idx, drain):
            b = s8 % 2
            pltpu.make_async_copy(dstr.at[pl.ds(ebase + j * ch, ch)],
                                  dsti.at[s8, 0], semi[s8 % 4]).wait()
            if drain:
                pltpu.make_async_copy(ones, acccnt.at[dsti.at[(s8 + 6) % 8, 0]],
                                      sems[b]).wait()
            pltpu.async_copy(ones, acccnt.at[dsti.at[s8, 0]], sems[b],
                             add=True)
            if pf_idx:
                issue_idx(j + 4, (s8 + 4) % 8)

        for j in range(4):
            issue_idx(j, j)

        tail_start = ((nchunk - 4) // 8) * 8
        for j in range(8):
            process(j, j, True, j >= 2)

        def step(g, carry):
            for b8 in range(8):
                process(8 * g + b8, b8, True, True)
            return carry
        lax.fori_loop(1, tail_start // 8, step, 0)
        for j in range(tail_start, nchunk):
            process(j, j % 8, j + 4 < nchunk, True)
        for j in (nchunk - 2, nchunk - 1):
            pltpu.make_async_copy(ones, acccnt.at[dsti.at[j % 8, 0]],
                                  sems[j % 2]).wait()

        plsc.subcore_barrier()
        pltpu.sync_copy(acccnt.at[pl.ds(base_row, rpt)],
                        out_c.at[c, pl.ds(base_row, rpt)])

    return pl.kernel(body, out_type=out_type, mesh=mesh,
                     scratch_types=scratch)


def _make_edge_update_kernel(nn, dp, ee, e0, esz):
    w = NC * NS
    ept = esz // w
    ch = 64
    nchunk = -(-ept // ch)
    assert ept % 8 == 0
    assert nchunk >= 8

    mesh = plsc.VectorSubcoreMesh(core_axis_name="c", subcore_axis_name="s",
                                  num_cores=NC, num_subcores=NS)
    out_type = jax.ShapeDtypeStruct((esz, dp), jnp.float32)
    scratch = (
        [pltpu.VMEM((8, 2, ch), jnp.int32),
         pltpu.VMEM((4, ch, dp), jnp.float32),
         pltpu.VMEM((4, ch, dp), jnp.float32),
         pltpu.VMEM((4, ch, dp), jnp.float32)]
        + [pltpu.SemaphoreType.DMA] * 12
    )

    def body(hij, eaw2, srcr, dstr, out, ibuf, g1, g2, eb, *sem):
        c = lax.axis_index("c")
        s = lax.axis_index("s")
        semi = list(sem[0:4])
        sem1 = list(sem[4:6])
        sem2 = list(sem[6:8])
        seme = list(sem[8:10])
        semo = list(sem[10:12])
        ebase = (c * NS + s) * ept
        gbase = e0 + ebase

        def coff(j):
            if isinstance(j, int):
                return min(j * ch, ept - ch)
            return jnp.minimum(j * ch, ept - ch)

        def issue_idx(j, s8):
            boff = gbase + coff(j)
            pltpu.async_copy(srcr.at[pl.ds(boff, ch)], ibuf.at[s8, 0],
                             semi[s8 % 4])
            pltpu.async_copy(dstr.at[pl.ds(boff, ch)], ibuf.at[s8, 1],
                             semi[s8 % 4])

        def wait_idx(j, s8):
            boff = gbase + coff(j)
            pltpu.make_async_copy(srcr.at[pl.ds(boff, ch)],
                                  ibuf.at[s8, 0], semi[s8 % 4]).wait()
            pltpu.make_async_copy(dstr.at[pl.ds(boff, ch)],
                                  ibuf.at[s8, 1], semi[s8 % 4]).wait()

        def issue_loads(j, s8):
            boff = gbase + coff(j)
            s4 = s8 % 4
            pltpu.async_copy(hij.at[ibuf.at[s8, 0]], g1.at[s4],
                             sem1[s8 % 2])
            pltpu.async_copy(hij.at[ibuf.at[s8, 1]], g2.at[s4],
                             sem2[s8 % 2])
            pltpu.async_copy(eaw2.at[pl.ds(boff, ch)], eb.at[s4],
                             seme[s8 % 2])

        def process(j, s8, pf_idx, pf_g, drain):
            b = s8 % 2
            s4 = s8 % 4
            boff = ebase + coff(j)
            goff = gbase + coff(j)
            pltpu.make_async_copy(hij.at[ibuf.at[s8, 0]], g1.at[s4],
                                  sem1[b]).wait()
            pltpu.make_async_copy(hij.at[ibuf.at[s8, 1]], g2.at[s4],
                                  sem2[b]).wait()
            pltpu.make_async_copy(eaw2.at[pl.ds(goff, ch)], eb.at[s4],
                                  seme[b]).wait()
            if drain:
                pltpu.make_async_copy(
                    eb.at[(s4 + 2) % 4],
                    out.at[pl.ds(ebase + coff(j - 2), ch)], semo[b]).wait()
            if pf_g:
                wait_idx(j + 2, (s8 + 2) % 8)
                issue_loads(j + 2, (s8 + 2) % 8)

            def crow(r, carry2):
                v = (eb[s4, r, 0:LANES] + g1[s4, r, 0:LANES]
                     + g2[s4, r, LANES:2 * LANES])
                eb[s4, r, 0:LANES] = jnp.maximum(v, 0.0)
                return carry2
            lax.fori_loop(0, ch, crow, 0)

            pltpu.async_copy(eb.at[s4], out.at[pl.ds(boff, ch)], semo[b])
            if pf_idx:
                issue_idx(j + 4, (s8 + 4) % 8)

        for j in range(4):
            issue_idx(j, j)
        for j in range(2):
            wait_idx(j, j)
            issue_loads(j, j)

        tail_start = ((nchunk - 4) // 8) * 8
        for j in range(8):
            process(j, j, True, True, j >= 2)

        def step(g, carry):
            for b8 in range(8):
                process(8 * g + b8, b8, True, True, True)
            return carry
        lax.fori_loop(1, tail_start // 8, step, 0)
        for j in range(tail_start, nchunk):
            process(j, j % 8, j + 4 < nchunk, j + 2 < nchunk, True)
        for j in (nchunk - 2, nchunk - 1):
            pltpu.make_async_copy(
                eb.at[j % 4], out.at[pl.ds(ebase + coff(j), ch)],
                semo[j % 2]).wait()

    return pl.kernel(body, out_type=out_type, mesh=mesh,
                     scratch_types=scratch)


def kernel(x, edge_attr, edge_index, Wm, bm, Wa, ba, We, be):
    n, d = x.shape
    e, de = edge_attr.shape
    nl = Wm.shape[0]
    assert de == LANES

    src = edge_index[0].astype(jnp.int32)
    dst = edge_index[1].astype(jnp.int32)

    bn = 400
    be_blk = 4000

    dp = 128

    msg = _make_msg_kernel(n, d, e)
    cntk = _make_cnt_kernel(n, e, d)
    edge_upd = _make_edge_update_kernel(n, dp, e, 0, e)

    h = x
    ea = edge_attr
    hx = _tc_node_matmul(x, Wm[0][:d], bn)
    cp = cntk(dst)
    for l in range(nl):
        last = l == nl - 1
        eaw = _edge_prep(ea, de, Wm[l][d:], bm[l][None], be_blk)
        if not last:
            wee_p = jnp.pad(We[l][2 * d:], ((0, 0), (0, dp - de)))
            bev_p = jnp.pad(be[l], (0, dp - de))[None]
            eaw2 = _edge_prep2(ea, de, wee_p, bev_p, be_blk)
        sp = msg(hx, eaw, src, dst)
        if not last:
            wij_p = jnp.pad(
                jnp.concatenate([We[l][:d], We[l][d:2 * d]], axis=1),
                ((0, 0), (0, dp - 2 * de)))
            h, hx, hij = _update(sp, cp, h, Wa[l][:d], Wa[l][d:],
                                 ba[l][None], Wm[l + 1][:d], wij_p, bn)
            ea = edge_upd(hij, eaw2, src, dst)
        else:
            h = _update(sp, cp, h, Wa[l][:d], Wa[l][d:], ba[l][None],
                        None, None, bn)
    return h


# speedup vs baseline: 1.1525x; 1.0419x over previous
import functools

import jax
import jax.numpy as jnp
from jax import lax
from jax.experimental import pallas as pl
from jax.experimental.pallas import tpu as pltpu
from jax.experimental.pallas import tpu_sc as plsc

NC = 2
NS = 16
LANES = 16


def _prep0_body(x_ref, w_ref, o_ref):
    o_ref[...] = jnp.dot(x_ref[...], w_ref[...],
                         preferred_element_type=jnp.float32)


def _tc_node_matmul(x, w, bn):
    n, d = x.shape
    return pl.pallas_call(
        _prep0_body,
        grid=(n // bn,),
        in_specs=[
            pl.BlockSpec((bn, d), lambda i: (i, 0)),
            pl.BlockSpec((d, w.shape[1]), lambda i: (0, 0)),
        ],
        out_specs=pl.BlockSpec((bn, w.shape[1]), lambda i: (i, 0)),
        out_shape=jax.ShapeDtypeStruct((n, w.shape[1]), jnp.float32),
    )(x, w)


def _edge_prep2_body(de, ea_ref, wee_ref, be_ref, eaw2_ref):
    ea = ea_ref[...][:, 0:de]
    eaw2_ref[...] = jnp.dot(ea, wee_ref[...],
                            preferred_element_type=jnp.float32) + be_ref[...]


def _edge_prep1_body(de, ea_ref, wme_ref, bm_ref, eaw_ref):
    ea = ea_ref[...][:, 0:de]
    eaw_ref[...] = jnp.dot(ea, wme_ref[...],
                           preferred_element_type=jnp.float32) + bm_ref[...]


def _edge_prep(ea, de, wme, bmv, be_blk):
    e, din = ea.shape
    d = wme.shape[1]
    return pl.pallas_call(
        functools.partial(_edge_prep1_body, de),
        grid=(e // be_blk,),
        in_specs=[
            pl.BlockSpec((be_blk, din), lambda i: (i, 0)),
            pl.BlockSpec((de, d), lambda i: (0, 0)),
            pl.BlockSpec((1, d), lambda i: (0, 0)),
        ],
        out_specs=pl.BlockSpec((be_blk, d), lambda i: (i, 0)),
        out_shape=jax.ShapeDtypeStruct((e, d), jnp.float32),
    )(ea, wme, bmv)


def _edge_prep1_alias_body(de, ea_ref, wme_ref, bm_ref, buf_ref, eaw_ref):
    del buf_ref
    ea = ea_ref[...][:, 0:de]
    eaw_ref[...] = jnp.dot(ea, wme_ref[...],
                           preferred_element_type=jnp.float32) + bm_ref[...]


def _edge_prep_half(ea_half, de, wme, bmv, be_blk, e_total, half_idx,
                    eaw_buf=None):
    eh, din = ea_half.shape
    d = wme.shape[1]
    nb = eh // be_blk
    off = half_idx * nb
    if eaw_buf is None:
        return pl.pallas_call(
            functools.partial(_edge_prep1_body, de),
            grid=(nb,),
            in_specs=[
                pl.BlockSpec((be_blk, din), lambda i: (i, 0)),
                pl.BlockSpec((de, d), lambda i: (0, 0)),
                pl.BlockSpec((1, d), lambda i: (0, 0)),
            ],
            out_specs=pl.BlockSpec((be_blk, d), lambda i: (i + off, 0)),
            out_shape=jax.ShapeDtypeStruct((e_total, d), jnp.float32),
        )(ea_half, wme, bmv)
    return pl.pallas_call(
        functools.partial(_edge_prep1_alias_body, de),
        grid=(nb,),
        in_specs=[
            pl.BlockSpec((be_blk, din), lambda i: (i, 0)),
            pl.BlockSpec((de, d), lambda i: (0, 0)),
            pl.BlockSpec((1, d), lambda i: (0, 0)),
            pl.BlockSpec((be_blk, d), lambda i: (i + off, 0)),
        ],
        out_specs=pl.BlockSpec((be_blk, d), lambda i: (i + off, 0)),
        out_shape=jax.ShapeDtypeStruct((e_total, d), jnp.float32),
        input_output_aliases={3: 0},
    )(ea_half, wme, bmv, eaw_buf)


def _edge_prep2_alias_body(de, ea_ref, wee_ref, be_ref, buf_ref, eaw2_ref):
    del buf_ref
    ea = ea_ref[...][:, 0:de]
    eaw2_ref[...] = jnp.dot(ea, wee_ref[...],
                            preferred_element_type=jnp.float32) + be_ref[...]


def _edge_prep2_half(ea_half, de, wee_p, bev_p, be_blk, e_total, half_idx,
                     buf=None):
    eh, din = ea_half.shape
    dp = wee_p.shape[1]
    nb = eh // be_blk
    off = half_idx * nb
    if buf is None:
        return pl.pallas_call(
            functools.partial(_edge_prep2_body, de),
            grid=(nb,),
            in_specs=[
                pl.BlockSpec((be_blk, din), lambda i: (i, 0)),
                pl.BlockSpec((de, dp), lambda i: (0, 0)),
                pl.BlockSpec((1, dp), lambda i: (0, 0)),
            ],
            out_specs=pl.BlockSpec((be_blk, dp), lambda i: (i + off, 0)),
            out_shape=jax.ShapeDtypeStruct((e_total, dp), jnp.float32),
        )(ea_half, wee_p, bev_p)
    return pl.pallas_call(
        functools.partial(_edge_prep2_alias_body, de),
        grid=(nb,),
        in_specs=[
            pl.BlockSpec((be_blk, din), lambda i: (i, 0)),
            pl.BlockSpec((de, dp), lambda i: (0, 0)),
            pl.BlockSpec((1, dp), lambda i: (0, 0)),
            pl.BlockSpec((be_blk, dp), lambda i: (i + off, 0)),
        ],
        out_specs=pl.BlockSpec((be_blk, dp), lambda i: (i + off, 0)),
        out_shape=jax.ShapeDtypeStruct((e_total, dp), jnp.float32),
        input_output_aliases={3: 0},
    )(ea_half, wee_p, bev_p, buf)


def _edge_prep2(ea, de, wee_p, bev_p, be_blk):
    e, din = ea.shape
    dp = wee_p.shape[1]
    return pl.pallas_call(
        functools.partial(_edge_prep2_body, de),
        grid=(e // be_blk,),
        in_specs=[
            pl.BlockSpec((be_blk, din), lambda i: (i, 0)),
            pl.BlockSpec((de, dp), lambda i: (0, 0)),
            pl.BlockSpec((1, dp), lambda i: (0, 0)),
        ],
        out_specs=pl.BlockSpec((be_blk, dp), lambda i: (i, 0)),
        out_shape=jax.ShapeDtypeStruct((e, dp), jnp.float32),
    )(ea, wee_p, bev_p)


def _update2_body(sp_ref, cp_ref, h_ref, waa_ref, wah_ref, ba_ref,
                  wmxn_ref, wij_ref,
                  hn_ref, hxn_ref, hij_ref):
    s = sp_ref[0] + sp_ref[1]
    cnt = cp_ref[0, :, 0:1] + cp_ref[1, :, 0:1]
    agg = s * (1.0 / jnp.maximum(cnt, 1.0))
    u = jnp.dot(agg, waa_ref[...], preferred_element_type=jnp.float32)
    u = u + jnp.dot(h_ref[...], wah_ref[...],
                    preferred_element_type=jnp.float32)
    u = jnp.maximum(u + ba_ref[...], 0.0)
    nn = jnp.sqrt(jnp.sum(u * u, axis=1, keepdims=True))
    hv = u / jnp.maximum(nn, 1e-12)
    hn_ref[...] = hv
    hxn_ref[...] = jnp.dot(hv, wmxn_ref[...],
                           preferred_element_type=jnp.float32)
    hij_ref[...] = jnp.dot(hv, wij_ref[...],
                           preferred_element_type=jnp.float32)


def _update1_body(sp_ref, cp_ref, h_ref, waa_ref, wah_ref, ba_ref, hn_ref):
    s = sp_ref[0] + sp_ref[1]
    cnt = cp_ref[0, :, 0:1] + cp_ref[1, :, 0:1]
    agg = s * (1.0 / jnp.maximum(cnt, 1.0))
    u = jnp.dot(agg, waa_ref[...], preferred_element_type=jnp.float32)
    u = u + jnp.dot(h_ref[...], wah_ref[...],
                    preferred_element_type=jnp.float32)
    u = jnp.maximum(u + ba_ref[...], 0.0)
    nn = jnp.sqrt(jnp.sum(u * u, axis=1, keepdims=True))
    hn_ref[...] = u / jnp.maximum(nn, 1e-12)


def _update(sp, cp, h, waa, wah, bav, wmxn, wij_p, bn):
    n, d = h.shape
    de = cp.shape[2]
    grid = (n // bn,)
    common_in = [
        pl.BlockSpec((NC, bn, d), lambda i: (0, i, 0)),
        pl.BlockSpec((NC, bn, de), lambda i: (0, i, 0)),
        pl.BlockSpec((bn, d), lambda i: (i, 0)),
        pl.BlockSpec((d, d), lambda i: (0, 0)),
        pl.BlockSpec((d, d), lambda i: (0, 0)),
        pl.BlockSpec((1, d), lambda i: (0, 0)),
    ]
    if wmxn is None:
        return pl.pallas_call(
            _update1_body,
            grid=grid,
            in_specs=common_in,
            out_specs=pl.BlockSpec((bn, d), lambda i: (i, 0)),
            out_shape=jax.ShapeDtypeStruct((n, d), jnp.float32),
        )(sp, cp, h, waa, wah, bav)
    dp = wij_p.shape[1]
    return pl.pallas_call(
        _update2_body,
        grid=grid,
        in_specs=common_in + [
            pl.BlockSpec((d, d), lambda i: (0, 0)),
            pl.BlockSpec((d, dp), lambda i: (0, 0)),
        ],
        out_specs=[
            pl.BlockSpec((bn, d), lambda i: (i, 0)),
            pl.BlockSpec((bn, d), lambda i: (i, 0)),
            pl.BlockSpec((bn, dp), lambda i: (i, 0)),
        ],
        out_shape=[
            jax.ShapeDtypeStruct((n, d), jnp.float32),
            jax.ShapeDtypeStruct((n, d), jnp.float32),
            jax.ShapeDtypeStruct((n, dp), jnp.float32),
        ],
    )(sp, cp, h, waa, wah, bav, wmxn, wij_p)


def _padded_rows(nn):
    rpt = -(-nn // NS)
    rpt = -(-rpt // 128) * 128
    return rpt, rpt * NS


def _make_msg_kernel(nn, dd, ee):
    w = NC * NS
    ept = ee // w
    ch = 40
    nchunk = ept // ch
    rpt, nnp = _padded_rows(nn)
    nz = rpt // ch
    assert ept % ch == 0 and rpt % ch == 0 and dd % LANES == 0
    assert nchunk % 2 == 0 and nchunk >= 8

    mesh = plsc.VectorSubcoreMesh(core_axis_name="c", subcore_axis_name="s",
                                  num_cores=NC, num_subcores=NS)

    out_type = jax.ShapeDtypeStruct((NC, nnp, dd), jnp.float32)
    scratch = (
        [pltpu.VMEM((8, 2, ch), jnp.int32),
         pltpu.VMEM((4, ch, dd), jnp.float32),
         pltpu.VMEM((2, ch, dd), jnp.float32),
         pltpu.VMEM_SHARED((nnp, dd), jnp.float32)]
        + [pltpu.SemaphoreType.DMA] * 10
    )

    def body(hx, eaw, srcr, dstr, out_s, ibuf, rows, eawb, acc, *sem):
        c = lax.axis_index("c")
        s = lax.axis_index("s")
        ncol = dd // LANES
        semi = list(sem[0:4])
        semg = list(sem[4:6])
        seme = list(sem[6:8])
        sems = list(sem[8:10])

        def zrow(r, carry):
            for cc in range(ncol):
                rows[0, r, cc * LANES:(cc + 1) * LANES] = jnp.zeros(
                    (LANES,), jnp.float32)
            return carry
        lax.fori_loop(0, ch, zrow, 0)

        base_row = s * rpt
        for z in range(nz):
            pltpu.sync_copy(rows.at[0], acc.at[pl.ds(base_row + z * ch, ch)])

        plsc.subcore_barrier()

        ebase = (c * NS + s) * ept

        def issue_idx(j, s8):
            boff = ebase + j * ch
            pltpu.async_copy(srcr.at[pl.ds(boff, ch)], ibuf.at[s8, 0],
                             semi[s8 % 4])
            pltpu.async_copy(dstr.at[pl.ds(boff, ch)], ibuf.at[s8, 1],
                             semi[s8 % 4])

        def wait_idx(j, s8):
            boff = ebase + j * ch
            pltpu.make_async_copy(srcr.at[pl.ds(boff, ch)], ibuf.at[s8, 0],
                                  semi[s8 % 4]).wait()
            pltpu.make_async_copy(dstr.at[pl.ds(boff, ch)], ibuf.at[s8, 1],
                                  semi[s8 % 4]).wait()

        def process(j, s8, pf_idx, pf_g, drain):
            b = s8 % 2
            s4 = s8 % 4
            boff = ebase + j * ch
            pltpu.make_async_copy(hx.at[ibuf.at[s8, 0]], rows.at[s4],
                                  semg[b]).wait()
            pltpu.make_async_copy(eaw.at[pl.ds(boff, ch)], eawb.at[b],
                                  seme[b]).wait()
            if drain:
                pltpu.make_async_copy(
                    rows.at[(s4 + 2) % 4], acc.at[ibuf.at[(s8 + 6) % 8, 1]],
                    sems[b]).wait()
            if pf_g:
                wait_idx(j + 2, (s8 + 2) % 8)
                pltpu.async_copy(hx.at[ibuf.at[(s8 + 2) % 8, 0]],
                                 rows.at[(s4 + 2) % 4], semg[b])

            def crow(r, carry2):
                for cc in range(ncol):
                    sl = slice(cc * LANES, (cc + 1) * LANES)
                    rows[s4, r, sl] = jnp.maximum(
                        rows[s4, r, sl] + eawb[b, r, sl], 0.0)
                return carry2
            lax.fori_loop(0, ch, crow, 0)

            pltpu.async_copy(rows.at[s4], acc.at[ibuf.at[s8, 1]],
                             sems[b], add=True)
            if pf_g:
                pltpu.async_copy(eaw.at[pl.ds(boff + 2 * ch, ch)],
                                 eawb.at[b], seme[b])
            if pf_idx:
                issue_idx(j + 4, (s8 + 4) % 8)

        for j in range(4):
            issue_idx(j, j)
        for j in range(2):
            wait_idx(j, j)
            pltpu.async_copy(hx.at[ibuf.at[j, 0]], rows.at[j], semg[j])
            pltpu.async_copy(eaw.at[pl.ds(ebase + j * ch, ch)],
                             eawb.at[j], seme[j])

        tail_start = ((nchunk - 4) // 8) * 8
        for j in range(8):
            process(j, j, True, True, j >= 2)

        def step(g, carry):
            for b8 in range(8):
                process(8 * g + b8, b8, True, True, True)
            return carry
        lax.fori_loop(1, tail_start // 8, step, 0)
        for j in range(tail_start, nchunk):
            process(j, j % 8, j + 4 < nchunk, j + 2 < nchunk, True)
        for j in (nchunk - 2, nchunk - 1):
            pltpu.make_async_copy(
                rows.at[j % 4], acc.at[ibuf.at[j % 8, 1]],
                sems[j % 2]).wait()

        plsc.subcore_barrier()
        pltpu.sync_copy(acc.at[pl.ds(base_row, rpt)],
                        out_s.at[c, pl.ds(base_row, rpt)])

    return pl.kernel(body, out_type=out_type, mesh=mesh,
                     scratch_types=scratch)


def _make_cnt_kernel(nn, ee, dd):
    w = NC * NS
    ept = ee // w
    ch = 40
    nchunk = ept // ch
    rpt, nnp = _padded_rows(nn)
    nz = rpt // ch
    assert ept % ch == 0 and rpt % ch == 0
    assert nchunk % 2 == 0 and nchunk >= 8

    mesh = plsc.VectorSubcoreMesh(core_axis_name="c", subcore_axis_name="s",
                                  num_cores=NC, num_subcores=NS)
    out_type = jax.ShapeDtypeStruct((NC, nnp, dd), jnp.float32)
    scratch = (
        [pltpu.VMEM((8, 1, ch), jnp.int32),
         pltpu.VMEM((ch, dd), jnp.float32),
         pltpu.VMEM((ch, dd), jnp.float32),
         pltpu.VMEM_SHARED((nnp, dd), jnp.float32)]
        + [pltpu.SemaphoreType.DMA] * 6
    )

    def body(dstr, out_c, dsti, ones, zbuf, acccnt, *sem):
        c = lax.axis_index("c")
        s = lax.axis_index("s")
        semi = list(sem[0:4])
        sems = list(sem[4:6])

        def fill(r, carry):
            for cc in range(dd // LANES):
                sl = slice(cc * LANES, (cc + 1) * LANES)
                ones[r, sl] = jnp.ones((LANES,), jnp.float32)
                zbuf[r, sl] = jnp.zeros((LANES,), jnp.float32)
            return carry
        lax.fori_loop(0, ch, fill, 0)

        base_row = s * rpt
        for z in range(nz):
            pltpu.sync_copy(zbuf, acccnt.at[pl.ds(base_row + z * ch, ch)])

        plsc.subcore_barrier()

        ebase = (c * NS + s) * ept

        def issue_idx(j, s8):
            pltpu.async_copy(dstr.at[pl.ds(ebase + j * ch, ch)],
                             dsti.at[s8, 0], semi[s8 % 4])

        def process(j, s8, pf_idx, drain):
            b = s8 % 2
            pltpu.make_async_copy(dstr.at[pl.ds(ebase + j * ch, ch)],
                                  dsti.at[s8, 0], semi[s8 % 4]).wait()
            if drain:
                pltpu.make_async_copy(ones, acccnt.at[dsti.at[(s8 + 6) % 8, 0]],
                                      sems[b]).wait()
            pltpu.async_copy(ones, acccnt.at[dsti.at[s8, 0]], sems[b],
                             add=True)
            if pf_idx:
                issue_idx(j + 4, (s8 + 4) % 8)

        for j in range(4):
            issue_idx(j, j)

        tail_start = ((nchunk - 4) // 8) * 8
        for j in range(8):
            process(j, j, True, j >= 2)

        def step(g, carry):
            for b8 in range(8):
                process(8 * g + b8, b8, True, True)
            return carry
        lax.fori_loop(1, tail_start // 8, step, 0)
        for j in range(tail_start, nchunk):
            process(j, j % 8, j + 4 < nchunk, True)
        for j in (nchunk - 2, nchunk - 1):
            pltpu.make_async_copy(ones, acccnt.at[dsti.at[j % 8, 0]],
                                  sems[j % 2]).wait()

        plsc.subcore_barrier()
        pltpu.sync_copy(acccnt.at[pl.ds(base_row, rpt)],
                        out_c.at[c, pl.ds(base_row, rpt)])

    return pl.kernel(body, out_type=out_type, mesh=mesh,
                     scratch_types=scratch)


def _make_edge_update_kernel(nn, dp, ee, e0, esz):
    w = NC * NS
    ept = esz // w
    ch = 64
    nchunk = -(-ept // ch)
    assert ept % 8 == 0
    assert nchunk >= 8

    mesh = plsc.VectorSubcoreMesh(core_axis_name="c", subcore_axis_name="s",
                                  num_cores=NC, num_subcores=NS)
    out_type = jax.ShapeDtypeStruct((esz, dp), jnp.float32)
    scratch = (
        [pltpu.VMEM((8, 2, ch), jnp.int32),
         pltpu.VMEM((4, ch, dp), jnp.float32),
         pltpu.VMEM((4, ch, dp), jnp.float32),
         pltpu.VMEM((4, ch, dp), jnp.float32)]
        + [pltpu.SemaphoreType.DMA] * 12
    )

    def body(hij, eaw2, srcr, dstr, out, ibuf, g1, g2, eb, *sem):
        c = lax.axis_index("c")
        s = lax.axis_index("s")
        semi = list(sem[0:4])
        sem1 = list(sem[4:6])
        sem2 = list(sem[6:8])
        seme = list(sem[8:10])
        semo = list(sem[10:12])
        ebase = (c * NS + s) * ept
        gbase = e0 + ebase

        def coff(j):
            if isinstance(j, int):
                return min(j * ch, ept - ch)
            return jnp.minimum(j * ch, ept - ch)

        def issue_idx(j, s8):
            boff = gbase + coff(j)
            pltpu.async_copy(srcr.at[pl.ds(boff, ch)], ibuf.at[s8, 0],
                             semi[s8 % 4])
            pltpu.async_copy(dstr.at[pl.ds(boff, ch)], ibuf.at[s8, 1],
                             semi[s8 % 4])

        def wait_idx(j, s8):
            boff = gbase + coff(j)
            pltpu.make_async_copy(srcr.at[pl.ds(boff, ch)],
                                  ibuf.at[s8, 0], semi[s8 % 4]).wait()
            pltpu.make_async_copy(dstr.at[pl.ds(boff, ch)],
                                  ibuf.at[s8, 1], semi[s8 % 4]).wait()

        def issue_loads(j, s8):
            boff = gbase + coff(j)
            s4 = s8 % 4
            pltpu.async_copy(hij.at[ibuf.at[s8, 0]], g1.at[s4],
                             sem1[s8 % 2])
            pltpu.async_copy(hij.at[ibuf.at[s8, 1]], g2.at[s4],
                             sem2[s8 % 2])
            pltpu.async_copy(eaw2.at[pl.ds(boff, ch)], eb.at[s4],
                             seme[s8 % 2])

        def process(j, s8, pf_idx, pf_g, drain):
            b = s8 % 2
            s4 = s8 % 4
            boff = ebase + coff(j)
            goff = gbase + coff(j)
            pltpu.make_async_copy(hij.at[ibuf.at[s8, 0]], g1.at[s4],
                                  sem1[b]).wait()
            pltpu.make_async_copy(hij.at[ibuf.at[s8, 1]], g2.at[s4],
                                  sem2[b]).wait()
            pltpu.make_async_copy(eaw2.at[pl.ds(goff, ch)], eb.at[s4],
                                  seme[b]).wait()
            if drain:
                pltpu.make_async_copy(
                    eb.at[(s4 + 2) % 4],
                    out.at[pl.ds(ebase + coff(j - 2), ch)], semo[b]).wait()
            if pf_g:
                wait_idx(j + 2, (s8 + 2) % 8)
                issue_loads(j + 2, (s8 + 2) % 8)

            def crow(r, carry2):
                v = (eb[s4, r, 0:LANES] + g1[s4, r, 0:LANES]
                     + g2[s4, r, LANES:2 * LANES])
                eb[s4, r, 0:LANES] = jnp.maximum(v, 0.0)
                return carry2
            lax.fori_loop(0, ch, crow, 0)

            pltpu.async_copy(eb.at[s4], out.at[pl.ds(boff, ch)], semo[b])
            if pf_idx:
                issue_idx(j + 4, (s8 + 4) % 8)

        for j in range(4):
            issue_idx(j, j)
        for j in range(2):
            wait_idx(j, j)
            issue_loads(j, j)

        tail_start = ((nchunk - 4) // 8) * 8
        for j in range(8):
            process(j, j, True, True, j >= 2)

        def step(g, carry):
            for b8 in range(8):
                process(8 * g + b8, b8, True, True, True)
            return carry
        lax.fori_loop(1, tail_start // 8, step, 0)
        for j in range(tail_start, nchunk):
            process(j, j % 8, j + 4 < nchunk, j + 2 < nchunk, True)
        for j in (nchunk - 2, nchunk - 1):
            pltpu.make_async_copy(
                eb.at[j % 4], out.at[pl.ds(ebase + coff(j), ch)],
                semo[j % 2]).wait()

    return pl.kernel(body, out_type=out_type, mesh=mesh,
                     scratch_types=scratch)


def kernel(x, edge_attr, edge_index, Wm, bm, Wa, ba, We, be):
    n, d = x.shape
    e, de = edge_attr.shape
    nl = Wm.shape[0]
    assert de == LANES

    src = edge_index[0].astype(jnp.int32)
    dst = edge_index[1].astype(jnp.int32)

    bn = 1000
    be_blk = 8000

    dp = 128

    msg = _make_msg_kernel(n, d, e)
    cntk = _make_cnt_kernel(n, e, d)
    edge_upd = _make_edge_update_kernel(n, dp, e, 0, e)

    h = x
    ea = edge_attr
    hx = _tc_node_matmul(x, Wm[0][:d], bn)
    cp = cntk(dst)
    for l in range(nl):
        last = l == nl - 1
        eaw = _edge_prep(ea, de, Wm[l][d:], bm[l][None], be_blk)
        if not last:
            wee_p = jnp.pad(We[l][2 * d:], ((0, 0), (0, dp - de)))
            bev_p = jnp.pad(be[l], (0, dp - de))[None]
            eaw2 = _edge_prep2(ea, de, wee_p, bev_p, be_blk)
        sp = msg(hx, eaw, src, dst)
        if not last:
            wij_p = jnp.pad(
                jnp.concatenate([We[l][:d], We[l][d:2 * d]], axis=1),
                ((0, 0), (0, dp - 2 * de)))
            h, hx, hij = _update(sp, cp, h, Wa[l][:d], Wa[l][d:],
                                 ba[l][None], Wm[l + 1][:d], wij_p, bn)
            ea = edge_upd(hij, eaw2, src, dst)
        else:
            h = _update(sp, cp, h, Wa[l][:d], Wa[l][d:], ba[l][None],
                        None, None, bn)
    return h


# speedup vs baseline: 1.1612x; 1.0076x over previous
import functools

import jax
import jax.numpy as jnp
from jax import lax
from jax.experimental import pallas as pl
from jax.experimental.pallas import tpu as pltpu
from jax.experimental.pallas import tpu_sc as plsc

NC = 2
NS = 16
LANES = 16


def _prep0_body(x_ref, w_ref, o_ref):
    o_ref[...] = jnp.dot(x_ref[...], w_ref[...],
                         preferred_element_type=jnp.float32)


def _tc_node_matmul(x, w, bn):
    n, d = x.shape
    return pl.pallas_call(
        _prep0_body,
        grid=(n // bn,),
        in_specs=[
            pl.BlockSpec((bn, d), lambda i: (i, 0)),
            pl.BlockSpec((d, w.shape[1]), lambda i: (0, 0)),
        ],
        out_specs=pl.BlockSpec((bn, w.shape[1]), lambda i: (i, 0)),
        out_shape=jax.ShapeDtypeStruct((n, w.shape[1]), jnp.float32),
    )(x, w)


def _edge_prep2_body(de, ea_ref, wee_ref, be_ref, eaw2_ref):
    ea = ea_ref[...][:, 0:de]
    eaw2_ref[...] = jnp.dot(ea, wee_ref[...],
                            preferred_element_type=jnp.float32) + be_ref[...]


def _edge_prep1_body(de, ea_ref, wme_ref, bm_ref, eaw_ref):
    ea = ea_ref[...][:, 0:de]
    eaw_ref[...] = jnp.dot(ea, wme_ref[...],
                           preferred_element_type=jnp.float32) + bm_ref[...]


def _edge_prep(ea, de, wme, bmv, be_blk):
    e, din = ea.shape
    d = wme.shape[1]
    return pl.pallas_call(
        functools.partial(_edge_prep1_body, de),
        grid=(e // be_blk,),
        in_specs=[
            pl.BlockSpec((be_blk, din), lambda i: (i, 0)),
            pl.BlockSpec((de, d), lambda i: (0, 0)),
            pl.BlockSpec((1, d), lambda i: (0, 0)),
        ],
        out_specs=pl.BlockSpec((be_blk, d), lambda i: (i, 0)),
        out_shape=jax.ShapeDtypeStruct((e, d), jnp.float32),
    )(ea, wme, bmv)


def _edge_prep1_alias_body(de, ea_ref, wme_ref, bm_ref, buf_ref, eaw_ref):
    del buf_ref
    ea = ea_ref[...][:, 0:de]
    eaw_ref[...] = jnp.dot(ea, wme_ref[...],
                           preferred_element_type=jnp.float32) + bm_ref[...]


def _edge_prep_half(ea_half, de, wme, bmv, be_blk, e_total, half_idx,
                    eaw_buf=None):
    eh, din = ea_half.shape
    d = wme.shape[1]
    nb = eh // be_blk
    off = half_idx * nb
    if eaw_buf is None:
        return pl.pallas_call(
            functools.partial(_edge_prep1_body, de),
            grid=(nb,),
            in_specs=[
                pl.BlockSpec((be_blk, din), lambda i: (i, 0)),
                pl.BlockSpec((de, d), lambda i: (0, 0)),
                pl.BlockSpec((1, d), lambda i: (0, 0)),
            ],
            out_specs=pl.BlockSpec((be_blk, d), lambda i: (i + off, 0)),
            out_shape=jax.ShapeDtypeStruct((e_total, d), jnp.float32),
        )(ea_half, wme, bmv)
    return pl.pallas_call(
        functools.partial(_edge_prep1_alias_body, de),
        grid=(nb,),
        in_specs=[
            pl.BlockSpec((be_blk, din), lambda i: (i, 0)),
            pl.BlockSpec((de, d), lambda i: (0, 0)),
            pl.BlockSpec((1, d), lambda i: (0, 0)),
            pl.BlockSpec((be_blk, d), lambda i: (i + off, 0)),
        ],
        out_specs=pl.BlockSpec((be_blk, d), lambda i: (i + off, 0)),
        out_shape=jax.ShapeDtypeStruct((e_total, d), jnp.float32),
        input_output_aliases={3: 0},
    )(ea_half, wme, bmv, eaw_buf)


def _edge_prep2_alias_body(de, ea_ref, wee_ref, be_ref, buf_ref, eaw2_ref):
    del buf_ref
    ea = ea_ref[...][:, 0:de]
    eaw2_ref[...] = jnp.dot(ea, wee_ref[...],
                            preferred_element_type=jnp.float32) + be_ref[...]


def _edge_prep2_half(ea_half, de, wee_p, bev_p, be_blk, e_total, half_idx,
                     buf=None):
    eh, din = ea_half.shape
    dp = wee_p.shape[1]
    nb = eh // be_blk
    off = half_idx * nb
    if buf is None:
        return pl.pallas_call(
            functools.partial(_edge_prep2_body, de),
            grid=(nb,),
            in_specs=[
                pl.BlockSpec((be_blk, din), lambda i: (i, 0)),
                pl.BlockSpec((de, dp), lambda i: (0, 0)),
                pl.BlockSpec((1, dp), lambda i: (0, 0)),
            ],
            out_specs=pl.BlockSpec((be_blk, dp), lambda i: (i + off, 0)),
            out_shape=jax.ShapeDtypeStruct((e_total, dp), jnp.float32),
        )(ea_half, wee_p, bev_p)
    return pl.pallas_call(
        functools.partial(_edge_prep2_alias_body, de),
        grid=(nb,),
        in_specs=[
            pl.BlockSpec((be_blk, din), lambda i: (i, 0)),
            pl.BlockSpec((de, dp), lambda i: (0, 0)),
            pl.BlockSpec((1, dp), lambda i: (0, 0)),
            pl.BlockSpec((be_blk, dp), lambda i: (i + off, 0)),
        ],
        out_specs=pl.BlockSpec((be_blk, dp), lambda i: (i + off, 0)),
        out_shape=jax.ShapeDtypeStruct((e_total, dp), jnp.float32),
        input_output_aliases={3: 0},
    )(ea_half, wee_p, bev_p, buf)


def _edge_prep2(ea, de, wee_p, bev_p, be_blk):
    e, din = ea.shape
    dp = wee_p.shape[1]
    return pl.pallas_call(
        functools.partial(_edge_prep2_body, de),
        grid=(e // be_blk,),
        in_specs=[
            pl.BlockSpec((be_blk, din), lambda i: (i, 0)),
            pl.BlockSpec((de, dp), lambda i: (0, 0)),
            pl.BlockSpec((1, dp), lambda i: (0, 0)),
        ],
        out_specs=pl.BlockSpec((be_blk, dp), lambda i: (i, 0)),
        out_shape=jax.ShapeDtypeStruct((e, dp), jnp.float32),
    )(ea, wee_p, bev_p)


def _update2_body(sp_ref, cp_ref, h_ref, waa_ref, wah_ref, ba_ref,
                  wmxn_ref, wij_ref,
                  hn_ref, hxn_ref, hij_ref):
    s = sp_ref[0] + sp_ref[1]
    cnt = cp_ref[0, :, 0:1] + cp_ref[1, :, 0:1]
    agg = s * (1.0 / jnp.maximum(cnt, 1.0))
    u = jnp.dot(agg, waa_ref[...], preferred_element_type=jnp.float32)
    u = u + jnp.dot(h_ref[...], wah_ref[...],
                    preferred_element_type=jnp.float32)
    u = jnp.maximum(u + ba_ref[...], 0.0)
    nn = jnp.sqrt(jnp.sum(u * u, axis=1, keepdims=True))
    hv = u / jnp.maximum(nn, 1e-12)
    hn_ref[...] = hv
    hxn_ref[...] = jnp.dot(hv, wmxn_ref[...],
                           preferred_element_type=jnp.float32)
    hij_ref[...] = jnp.dot(hv, wij_ref[...],
                           preferred_element_type=jnp.float32)


def _update1_body(sp_ref, cp_ref, h_ref, waa_ref, wah_ref, ba_ref, hn_ref):
    s = sp_ref[0] + sp_ref[1]
    cnt = cp_ref[0, :, 0:1] + cp_ref[1, :, 0:1]
    agg = s * (1.0 / jnp.maximum(cnt, 1.0))
    u = jnp.dot(agg, waa_ref[...], preferred_element_type=jnp.float32)
    u = u + jnp.dot(h_ref[...], wah_ref[...],
                    preferred_element_type=jnp.float32)
    u = jnp.maximum(u + ba_ref[...], 0.0)
    nn = jnp.sqrt(jnp.sum(u * u, axis=1, keepdims=True))
    hn_ref[...] = u / jnp.maximum(nn, 1e-12)


def _update(sp, cp, h, waa, wah, bav, wmxn, wij_p, bn):
    n, d = h.shape
    de = cp.shape[2]
    grid = (n // bn,)
    common_in = [
        pl.BlockSpec((NC, bn, d), lambda i: (0, i, 0)),
        pl.BlockSpec((NC, bn, de), lambda i: (0, i, 0)),
        pl.BlockSpec((bn, d), lambda i: (i, 0)),
        pl.BlockSpec((d, d), lambda i: (0, 0)),
        pl.BlockSpec((d, d), lambda i: (0, 0)),
        pl.BlockSpec((1, d), lambda i: (0, 0)),
    ]
    if wmxn is None:
        return pl.pallas_call(
            _update1_body,
            grid=grid,
            in_specs=common_in,
            out_specs=pl.BlockSpec((bn, d), lambda i: (i, 0)),
            out_shape=jax.ShapeDtypeStruct((n, d), jnp.float32),
        )(sp, cp, h, waa, wah, bav)
    dp = wij_p.shape[1]
    return pl.pallas_call(
        _update2_body,
        grid=grid,
        in_specs=common_in + [
            pl.BlockSpec((d, d), lambda i: (0, 0)),
            pl.BlockSpec((d, dp), lambda i: (0, 0)),
        ],
        out_specs=[
            pl.BlockSpec((bn, d), lambda i: (i, 0)),
            pl.BlockSpec((bn, d), lambda i: (i, 0)),
            pl.BlockSpec((bn, dp), lambda i: (i, 0)),
        ],
        out_shape=[
            jax.ShapeDtypeStruct((n, d), jnp.float32),
            jax.ShapeDtypeStruct((n, d), jnp.float32),
            jax.ShapeDtypeStruct((n, dp), jnp.float32),
        ],
    )(sp, cp, h, waa, wah, bav, wmxn, wij_p)


def _padded_rows(nn):
    rpt = -(-nn // NS)
    rpt = -(-rpt // 128) * 128
    return rpt, rpt * NS


def _make_msg_kernel(nn, dd, ee):
    w = NC * NS
    ept = ee // w
    ch = 40
    nchunk = ept // ch
    rpt, nnp = _padded_rows(nn)
    nz = rpt // ch
    assert ept % ch == 0 and rpt % ch == 0 and dd % LANES == 0
    assert nchunk % 2 == 0 and nchunk >= 8

    mesh = plsc.VectorSubcoreMesh(core_axis_name="c", subcore_axis_name="s",
                                  num_cores=NC, num_subcores=NS)

    out_type = jax.ShapeDtypeStruct((NC, nnp, dd), jnp.float32)
    scratch = (
        [pltpu.VMEM((8, 2, ch), jnp.int32),
         pltpu.VMEM((4, ch, dd), jnp.float32),
         pltpu.VMEM((2, ch, dd), jnp.float32),
         pltpu.VMEM_SHARED((nnp, dd), jnp.float32)]
        + [pltpu.SemaphoreType.DMA] * 10
    )

    def body(hx, eaw, srcr, dstr, out_s, ibuf, rows, eawb, acc, *sem):
        c = lax.axis_index("c")
        s = lax.axis_index("s")
        ncol = dd // LANES
        semi = list(sem[0:4])
        semg = list(sem[4:6])
        seme = list(sem[6:8])
        sems = list(sem[8:10])

        def zrow(r, carry):
            for cc in range(ncol):
                rows[0, r, cc * LANES:(cc + 1) * LANES] = jnp.zeros(
                    (LANES,), jnp.float32)
            return carry
        lax.fori_loop(0, ch, zrow, 0)

        base_row = s * rpt
        for z in range(nz):
            pltpu.sync_copy(rows.at[0], acc.at[pl.ds(base_row + z * ch, ch)])

        plsc.subcore_barrier()

        ebase = (c * NS + s) * ept

        def issue_idx(j, s8):
            boff = ebase + j * ch
            pltpu.async_copy(srcr.at[pl.ds(boff, ch)], ibuf.at[s8, 0],
                             semi[s8 % 4])
            pltpu.async_copy(dstr.at[pl.ds(boff, ch)], ibuf.at[s8, 1],
                             semi[s8 % 4])

        def wait_idx(j, s8):
            boff = ebase + j * ch
            pltpu.make_async_copy(srcr.at[pl.ds(boff, ch)], ibuf.at[s8, 0],
                                  semi[s8 % 4]).wait()
            pltpu.make_async_copy(dstr.at[pl.ds(boff, ch)], ibuf.at[s8, 1],
                                  semi[s8 % 4]).wait()

        def process(j, s8, pf_idx, pf_g, drain):
            b = s8 % 2
            s4 = s8 % 4
            boff = ebase + j * ch
            pltpu.make_async_copy(hx.at[ibuf.at[s8, 0]], rows.at[s4],
                                  semg[b]).wait()
            pltpu.make_async_copy(eaw.at[pl.ds(boff, ch)], eawb.at[b],
                                  seme[b]).wait()
            if drain:
                pltpu.make_async_copy(
                    rows.at[(s4 + 2) % 4], acc.at[ibuf.at[(s8 + 6) % 8, 1]],
                    sems[b]).wait()
            if pf_g:
                wait_idx(j + 2, (s8 + 2) % 8)
                pltpu.async_copy(hx.at[ibuf.at[(s8 + 2) % 8, 0]],
                                 rows.at[(s4 + 2) % 4], semg[b])

            def crow(r, carry2):
                for cc in range(ncol):
                    sl = slice(cc * LANES, (cc + 1) * LANES)
                    rows[s4, r, sl] = jnp.maximum(
                        rows[s4, r, sl] + eawb[b, r, sl], 0.0)
                return carry2
            lax.fori_loop(0, ch, crow, 0)

            pltpu.async_copy(rows.at[s4], acc.at[ibuf.at[s8, 1]],
                             sems[b], add=True)
            if pf_g:
                pltpu.async_copy(eaw.at[pl.ds(boff + 2 * ch, ch)],
                                 eawb.at[b], seme[b])
            if pf_idx:
                issue_idx(j + 4, (s8 + 4) % 8)

        for j in range(4):
            issue_idx(j, j)
        for j in range(2):
            wait_idx(j, j)
            pltpu.async_copy(hx.at[ibuf.at[j, 0]], rows.at[j], semg[j])
            pltpu.async_copy(eaw.at[pl.ds(ebase + j * ch, ch)],
                             eawb.at[j], seme[j])

        tail_start = ((nchunk - 4) // 8) * 8
        for j in range(8):
            process(j, j, True, True, j >= 2)

        def step(g, carry):
            for b8 in range(8):
                process(8 * g + b8, b8, True, True, True)
            return carry
        lax.fori_loop(1, tail_start // 8, step, 0)
        for j in range(tail_start, nchunk):
            process(j, j % 8, j + 4 < nchunk, j + 2 < nchunk, True)
        for j in (nchunk - 2, nchunk - 1):
            pltpu.make_async_copy(
                rows.at[j % 4], acc.at[ibuf.at[j % 8, 1]],
                sems[j % 2]).wait()

        plsc.subcore_barrier()
        pltpu.sync_copy(acc.at[pl.ds(base_row, rpt)],
                        out_s.at[c, pl.ds(base_row, rpt)])

    return pl.kernel(body, out_type=out_type, mesh=mesh,
                     scratch_types=scratch)


def _make_cnt_kernel(nn, ee, dd):
    w = NC * NS
    ept = ee // w
    ch = 40
    nchunk = ept // ch
    rpt, nnp = _padded_rows(nn)
    nz = rpt // ch
    assert ept % ch == 0 and rpt % ch == 0
    assert nchunk % 2 == 0 and nchunk >= 8

    mesh = plsc.VectorSubcoreMesh(core_axis_name="c", subcore_axis_name="s",
                                  num_cores=NC, num_subcores=NS)
    out_type = jax.ShapeDtypeStruct((NC, nnp, dd), jnp.float32)
    scratch = (
        [pltpu.VMEM((8, 1, ch), jnp.int32),
         pltpu.VMEM((ch, dd), jnp.float32),
         pltpu.VMEM((ch, dd), jnp.float32),
         pltpu.VMEM_SHARED((nnp, dd), jnp.float32)]
        + [pltpu.SemaphoreType.DMA] * 6
    )

    def body(dstr, out_c, dsti, ones, zbuf, acccnt, *sem):
        c = lax.axis_index("c")
        s = lax.axis_index("s")
        semi = list(sem[0:4])
        sems = list(sem[4:6])

        def fill(r, carry):
            for cc in range(dd // LANES):
                sl = slice(cc * LANES, (cc + 1) * LANES)
                ones[r, sl] = jnp.ones((LANES,), jnp.float32)
                zbuf[r, sl] = jnp.zeros((LANES,), jnp.float32)
            return carry
        lax.fori_loop(0, ch, fill, 0)

        base_row = s * rpt
        for z in range(nz):
            pltpu.sync_copy(zbuf, acccnt.at[pl.ds(base_row + z * ch, ch)])

        plsc.subcore_barrier()

        ebase = (c * NS + s) * ept

        def issue_idx(j, s8):
            pltpu.async_copy(dstr.at[pl.ds(ebase + j * ch, ch)],
                             dsti.at[s8, 0], semi[s8 % 4])

        def process(j, s8, pf_idx, drain):
            b = s8 % 2
            pltpu.make_async_copy(dstr.at[pl.ds(ebase + j * ch, ch)],
                                  dsti.at[s8, 0], semi[s8 % 4]).wait()
            if drain:
                pltpu.make_async_copy(ones, acccnt.at[dsti.at[(s8 + 6) % 8, 0]],
                                      sems[b]).wait()
            pltpu.async_copy(ones, acccnt.at[dsti.at[s8, 0]], sems[b],
                             add=True)
            if pf_idx:
                issue_idx(j + 4, (s8 + 4) % 8)

        for j in range(4):
            issue_idx(j, j)

        tail_start = ((nchunk - 4) // 8) * 8
        for j in range(8):
            process(j, j, True, j >= 2)

        def step(g, carry):
            for b8 in range(8):
                process(8 * g + b8, b8, True, True)
            return carry
        lax.fori_loop(1, tail_start // 8, step, 0)
        for j in range(tail_start, nchunk):
            process(j, j % 8, j + 4 < nchunk, True)
        for j in (nchunk - 2, nchunk - 1):
            pltpu.make_async_copy(ones, acccnt.at[dsti.at[j % 8, 0]],
                                  sems[j % 2]).wait()

        plsc.subcore_barrier()
        pltpu.sync_copy(acccnt.at[pl.ds(base_row, rpt)],
                        out_c.at[c, pl.ds(base_row, rpt)])

    return pl.kernel(body, out_type=out_type, mesh=mesh,
                     scratch_types=scratch)


def _make_edge_update_kernel(nn, dp, ee, e0, esz):
    w = NC * NS
    ept = esz // w
    ch = 64
    nchunk = -(-ept // ch)
    assert ept % 8 == 0
    assert nchunk >= 8

    mesh = plsc.VectorSubcoreMesh(core_axis_name="c", subcore_axis_name="s",
                                  num_cores=NC, num_subcores=NS)
    out_type = jax.ShapeDtypeStruct((esz, dp), jnp.float32)
    scratch = (
        [pltpu.VMEM((8, 2, ch), jnp.int32),
         pltpu.VMEM((4, ch, dp), jnp.float32),
         pltpu.VMEM((4, ch, dp), jnp.float32),
         pltpu.VMEM((4, ch, dp), jnp.float32)]
        + [pltpu.SemaphoreType.DMA] * 12
    )

    def body(hij, eaw2, srcr, dstr, out, ibuf, g1, g2, eb, *sem):
        c = lax.axis_index("c")
        s = lax.axis_index("s")
        semi = list(sem[0:4])
        sem1 = list(sem[4:6])
        sem2 = list(sem[6:8])
        seme = list(sem[8:10])
        semo = list(sem[10:12])
        ebase = (c * NS + s) * ept
        gbase = e0 + ebase

        def coff(j):
            if isinstance(j, int):
                return min(j * ch, ept - ch)
            return jnp.minimum(j * ch, ept - ch)

        def issue_idx(j, s8):
            boff = gbase + coff(j)
            pltpu.async_copy(srcr.at[pl.ds(boff, ch)], ibuf.at[s8, 0],
                             semi[s8 % 4])
            pltpu.async_copy(dstr.at[pl.ds(boff, ch)], ibuf.at[s8, 1],
                             semi[s8 % 4])

        def wait_idx(j, s8):
            boff = gbase + coff(j)
            pltpu.make_async_copy(srcr.at[pl.ds(boff, ch)],
                                  ibuf.at[s8, 0], semi[s8 % 4]).wait()
            pltpu.make_async_copy(dstr.at[pl.ds(boff, ch)],
                                  ibuf.at[s8, 1], semi[s8 % 4]).wait()

        def issue_loads(j, s8):
            boff = gbase + coff(j)
            s4 = s8 % 4
            pltpu.async_copy(hij.at[ibuf.at[s8, 0]], g1.at[s4],
                             sem1[s8 % 2])
            pltpu.async_copy(hij.at[ibuf.at[s8, 1]], g2.at[s4],
                             sem2[s8 % 2])
            pltpu.async_copy(eaw2.at[pl.ds(boff, ch)], eb.at[s4],
                             seme[s8 % 2])

        def process(j, s8, pf_idx, pf_g, drain):
            b = s8 % 2
            s4 = s8 % 4
            boff = ebase + coff(j)
            goff = gbase + coff(j)
            pltpu.make_async_copy(hij.at[ibuf.at[s8, 0]], g1.at[s4],
                                  sem1[b]).wait()
            pltpu.make_async_copy(hij.at[ibuf.at[s8, 1]], g2.at[s4],
                                  sem2[b]).wait()
            pltpu.make_async_copy(eaw2.at[pl.ds(goff, ch)], eb.at[s4],
                                  seme[b]).wait()
            if drain:
                pltpu.make_async_copy(
                    eb.at[(s4 + 2) % 4],
                    out.at[pl.ds(ebase + coff(j - 2), ch)], semo[b]).wait()
            if pf_g:
                wait_idx(j + 2, (s8 + 2) % 8)
                issue_loads(j + 2, (s8 + 2) % 8)

            def crow(r, carry2):
                v = (eb[s4, r, 0:LANES] + g1[s4, r, 0:LANES]
                     + g2[s4, r, LANES:2 * LANES])
                eb[s4, r, 0:LANES] = jnp.maximum(v, 0.0)
                return carry2
            lax.fori_loop(0, ch, crow, 0)

            pltpu.async_copy(eb.at[s4], out.at[pl.ds(boff, ch)], semo[b])
            if pf_idx:
                issue_idx(j + 4, (s8 + 4) % 8)

        for j in range(4):
            issue_idx(j, j)
        for j in range(2):
            wait_idx(j, j)
            issue_loads(j, j)

        tail_start = ((nchunk - 4) // 8) * 8
        for j in range(8):
            process(j, j, True, True, j >= 2)

        def step(g, carry):
            for b8 in range(8):
                process(8 * g + b8, b8, True, True, True)
            return carry
        lax.fori_loop(1, tail_start // 8, step, 0)
        for j in range(tail_start, nchunk):
            process(j, j % 8, j + 4 < nchunk, j + 2 < nchunk, True)
        for j in (nchunk - 2, nchunk - 1):
            pltpu.make_async_copy(
                eb.at[j % 4], out.at[pl.ds(ebase + coff(j), ch)],
                semo[j % 2]).wait()

    return pl.kernel(body, out_type=out_type, mesh=mesh,
                     scratch_types=scratch)


def kernel(x, edge_attr, edge_index, Wm, bm, Wa, ba, We, be):
    n, d = x.shape
    e, de = edge_attr.shape
    nl = Wm.shape[0]
    assert de == LANES

    src = edge_index[0].astype(jnp.int32)
    dst = edge_index[1].astype(jnp.int32)

    bn = 2000
    be_blk = 16000

    dp = 128

    msg = _make_msg_kernel(n, d, e)
    cntk = _make_cnt_kernel(n, e, d)
    edge_upd = _make_edge_update_kernel(n, dp, e, 0, e)

    h = x
    ea = edge_attr
    hx = _tc_node_matmul(x, Wm[0][:d], bn)
    cp = cntk(dst)
    for l in range(nl):
        last = l == nl - 1
        eaw = _edge_prep(ea, de, Wm[l][d:], bm[l][None], be_blk)
        if not last:
            wee_p = jnp.pad(We[l][2 * d:], ((0, 0), (0, dp - de)))
            bev_p = jnp.pad(be[l], (0, dp - de))[None]
            eaw2 = _edge_prep2(ea, de, wee_p, bev_p, be_blk)
        sp = msg(hx, eaw, src, dst)
        if not last:
            wij_p = jnp.pad(
                jnp.concatenate([We[l][:d], We[l][d:2 * d]], axis=1),
                ((0, 0), (0, dp - 2 * de)))
            h, hx, hij = _update(sp, cp, h, Wa[l][:d], Wa[l][d:],
                                 ba[l][None], Wm[l + 1][:d], wij_p, bn)
            ea = edge_upd(hij, eaw2, src, dst)
        else:
            h = _update(sp, cp, h, Wa[l][:d], Wa[l][d:], ba[l][None],
                        None, None, bn)
    return h
